# Initial kernel scaffold; baseline (speedup 1.0000x reference)
import jax
import jax.numpy as jnp
from jax.experimental import pallas as pl


def kernel(batch_box_preds, batch_cls_preds):
    raise NotImplementedError("write your pallas kernel here")


# speedup vs baseline: 26.9850x; 26.9850x over previous
import functools
import jax
import jax.numpy as jnp
from jax.experimental import pallas as pl
from jax.experimental.pallas import tpu as pltpu

SCORE_THRESH = 0.1
NMS_THRESH = 0.01
K = 4096
OUT = 500
NPAD = 20480
TB = 256
RCHUNK = 2048
NCHUNK = 1024


def _scores_body(cls_ref, masked_ref, label_ref):
    c0 = cls_ref[0, 0:1, :]
    c1 = cls_ref[0, 1:2, :]
    c2 = cls_ref[0, 2:3, :]
    s0 = jax.nn.sigmoid(c0)
    s1 = jax.nn.sigmoid(c1)
    s2 = jax.nn.sigmoid(c2)
    m01 = jnp.maximum(s0, s1)
    i01 = jnp.where(s1 > s0, 1, 0)
    score = jnp.maximum(m01, s2)
    label = jnp.where(s2 > m01, 2, i01) + 1
    masked = jnp.where(score >= SCORE_THRESH, score, -1.0)
    masked_ref[0, :, :] = masked
    label_ref[0, :, :] = label


def _stage1(cls3d):
    B = cls3d.shape[0]
    return pl.pallas_call(
        _scores_body,
        grid=(B,),
        in_specs=[pl.BlockSpec((1, 3, NPAD), lambda b: (b, 0, 0))],
        out_specs=[pl.BlockSpec((1, 1, NPAD), lambda b: (b, 0, 0)),
                   pl.BlockSpec((1, 1, NPAD), lambda b: (b, 0, 0))],
        out_shape=[jax.ShapeDtypeStruct((B, 1, NPAD), jnp.float32),
                   jax.ShapeDtypeStruct((B, 1, NPAD), jnp.int32)],
    )(cls3d)


def _rank_body(keyc_ref, keyr_ref, rank_ref):
    ib = pl.program_id(1)
    ki = keyc_ref[0]
    ii = ib * TB + jax.lax.broadcasted_iota(jnp.int32, (TB, 1), 0)

    def chunk(c, acc):
        kj = keyr_ref[0, :, pl.ds(c * RCHUNK, RCHUNK)]
        jj = c * RCHUNK + jax.lax.broadcasted_iota(jnp.int32, (1, RCHUNK), 1)
        cmp = (kj > ki) | ((kj == ki) & (jj < ii))
        return acc + jnp.sum(cmp.astype(jnp.int32), axis=1, keepdims=True)

    acc = jax.lax.fori_loop(0, NPAD // RCHUNK, chunk,
                            jnp.zeros((TB, 1), jnp.int32))
    rank_ref[0] = acc


def _stage2(key):
    B = key.shape[0]
    keyc = key.reshape(B, NPAD, 1)
    keyr = key.reshape(B, 1, NPAD)
    out = pl.pallas_call(
        _rank_body,
        grid=(B, NPAD // TB),
        in_specs=[pl.BlockSpec((1, TB, 1), lambda b, ib: (b, ib, 0)),
                  pl.BlockSpec((1, 1, NPAD), lambda b, ib: (b, 0, 0))],
        out_specs=pl.BlockSpec((1, TB, 1), lambda b, ib: (b, ib, 0)),
        out_shape=jax.ShapeDtypeStruct((B, NPAD, 1), jnp.int32),
    )(keyc, keyr)
    return out.reshape(B, NPAD)


def _nms_body(icols_ref, tblT_ref, keep_ref, alive_s, mloc_s):
    kb = pl.program_id(1)
    i0 = kb * TB
    lane1 = jax.lax.broadcasted_iota(jnp.int32, (1, NCHUNK), 1)
    laneT = jax.lax.broadcasted_iota(jnp.int32, (1, TB), 1)

    @pl.when(kb == 0)
    def _init():
        alive_s[0:1, :] = (tblT_ref[0, 7:8, :] > 0.0).astype(jnp.float32)

    xi = icols_ref[0, :, 0:1]
    yi = icols_ref[0, :, 1:2]
    zi = icols_ref[0, :, 2:3]
    hxi = jnp.abs(icols_ref[0, :, 3:4]) * 0.5
    hyi = jnp.abs(icols_ref[0, :, 4:5]) * 0.5
    hzi = jnp.abs(icols_ref[0, :, 5:6]) * 0.5
    voli = hxi * hyi * hzi * 8.0

    def iou_tile(xj, yj, zj, hxj, hyj, hzj, volj):
        ix = jnp.maximum(jnp.minimum(xi + hxi, xj + hxj) - jnp.maximum(xi - hxi, xj - hxj), 0.0)
        iy = jnp.maximum(jnp.minimum(yi + hyi, yj + hyj) - jnp.maximum(yi - hyi, yj - hyj), 0.0)
        iz = jnp.maximum(jnp.minimum(zi + hzi, zj + hzj) - jnp.maximum(zi - hzi, zj - hzj), 0.0)
        inter = ix * iy * iz
        return inter / (voli + volj - inter + 1e-8)

    def planes(j0, w):
        xj = tblT_ref[0, 0:1, pl.ds(j0, w)]
        yj = tblT_ref[0, 1:2, pl.ds(j0, w)]
        zj = tblT_ref[0, 2:3, pl.ds(j0, w)]
        hxj = jnp.abs(tblT_ref[0, 3:4, pl.ds(j0, w)]) * 0.5
        hyj = jnp.abs(tblT_ref[0, 4:5, pl.ds(j0, w)]) * 0.5
        hzj = jnp.abs(tblT_ref[0, 5:6, pl.ds(j0, w)]) * 0.5
        return xj, yj, zj, hxj, hyj, hzj, hxj * hyj * hzj * 8.0

    mloc_s[:, :] = (iou_tile(*planes(i0, TB)) > NMS_THRESH).astype(jnp.float32)
    al = alive_s[0:1, pl.ds(i0, TB)]

    def scan_i(i, al):
        row = mloc_s[pl.ds(i, 1), :]
        ai = jnp.sum(jnp.where(laneT == i, al, 0.0))
        sup = (row > 0.5) & (laneT > i) & (ai > 0.0)
        return jnp.where(sup, 0.0, al)

    al = jax.lax.fori_loop(0, TB, scan_i, al)
    alive_s[0:1, pl.ds(i0, TB)] = al

    def chunk(c, _):
        j0 = c * NCHUNK

        @pl.when(j0 + NCHUNK > i0 + TB)
        def _():
            Mc = (iou_tile(*planes(j0, NCHUNK)) > NMS_THRESH).astype(jnp.float32)
            sup = jnp.dot(al, Mc, preferred_element_type=jnp.float32)
            later = (j0 + lane1) >= i0 + TB
            hit = (sup > 0.5) & later
            cur = alive_s[0:1, pl.ds(j0, NCHUNK)]
            alive_s[0:1, pl.ds(j0, NCHUNK)] = jnp.where(hit, 0.0, cur)
        return 0

    jax.lax.fori_loop(0, K // NCHUNK, chunk, 0)
    keep_ref[0, 0:1, :] = alive_s[0:1, :]


def _stage4(icols, tblT):
    B = icols.shape[0]
    return pl.pallas_call(
        _nms_body,
        grid=(B, K // TB),
        in_specs=[pl.BlockSpec((1, TB, 8), lambda b, kb: (b, kb, 0)),
                  pl.BlockSpec((1, 8, K), lambda b, kb: (b, 0, 0))],
        out_specs=pl.BlockSpec((1, 1, K), lambda b, kb: (b, 0, 0)),
        out_shape=jax.ShapeDtypeStruct((B, 1, K), jnp.float32),
        scratch_shapes=[pltpu.VMEM((1, K), jnp.float32),
                        pltpu.VMEM((TB, TB), jnp.float32)],
    )(icols, tblT)


@jax.jit
def _run(batch_box_preds, batch_cls_preds):
    B, n, _ = batch_box_preds.shape
    cls3d = jnp.pad(batch_cls_preds, ((0, 0), (0, NPAD - n), (0, 0)),
                    constant_values=-30.0).transpose(0, 2, 1)
    boxes = jnp.pad(batch_box_preds, ((0, 0), (0, NPAD - n), (0, 0)))

    masked3, label3 = _stage1(cls3d)
    masked = masked3.reshape(B, NPAD)
    label = label3.reshape(B, NPAD)
    key = jax.lax.bitcast_convert_type(masked, jnp.int32)
    rank = _stage2(key)

    iidx = jnp.arange(NPAD, dtype=jnp.int32)
    top_idx = jnp.zeros((B, K), jnp.int32).at[
        jnp.arange(B)[:, None], rank].set(iidx[None, :], mode="drop")
    top_boxes = jnp.take_along_axis(boxes, top_idx[..., None], axis=1)
    top_scores = jnp.take_along_axis(masked, top_idx, axis=1)
    top_labels = jnp.take_along_axis(label, top_idx, axis=1)
    icols = jnp.concatenate([top_boxes, top_scores[..., None]], axis=-1)
    tblT = icols.transpose(0, 2, 1)

    keep3 = _stage4(icols, tblT)
    keep = keep3.reshape(B, K) > 0.5

    count = jnp.sum(keep, axis=1)
    pos = jnp.cumsum(keep.astype(jnp.int32), axis=1) - 1
    wpos = jnp.where(keep & (pos < OUT), pos, OUT)
    sel = jnp.zeros((B, OUT), jnp.int32).at[
        jnp.arange(B)[:, None], wpos].set(jnp.arange(K, dtype=jnp.int32)[None, :],
                                          mode="drop")
    out_valid = jnp.arange(OUT)[None, :] < count[:, None]
    fb = jnp.take_along_axis(top_boxes, sel[..., None], axis=1) * out_valid[..., None]
    fs = jnp.take_along_axis(top_scores, sel, axis=1) * out_valid
    fl = jnp.where(out_valid, jnp.take_along_axis(top_labels, sel, axis=1), 0)
    return fb, fs, fl


def kernel(batch_box_preds, batch_cls_preds):
    return _run(batch_box_preds, batch_cls_preds)


# speedup vs baseline: 28.0264x; 1.0386x over previous
import functools
import jax
import jax.numpy as jnp
from jax import lax
from jax.experimental import pallas as pl
from jax.experimental.pallas import tpu as pltpu
from jax.experimental.pallas import tpu_sc as plsc

SCORE_THRESH = 0.1
NMS_THRESH = 0.01
K = 4096
OUT = 500
NPAD = 20480
TB = 256
RCHUNK = 2048
NCHUNK = 1024


def _scores_body(cls_ref, masked_ref, label_ref):
    c0 = cls_ref[0, 0:1, :]
    c1 = cls_ref[0, 1:2, :]
    c2 = cls_ref[0, 2:3, :]
    s0 = jax.nn.sigmoid(c0)
    s1 = jax.nn.sigmoid(c1)
    s2 = jax.nn.sigmoid(c2)
    m01 = jnp.maximum(s0, s1)
    i01 = jnp.where(s1 > s0, 1, 0)
    score = jnp.maximum(m01, s2)
    label = jnp.where(s2 > m01, 2, i01) + 1
    masked = jnp.where(score >= SCORE_THRESH, score, -1.0)
    masked_ref[0, :, :] = masked
    label_ref[0, :, :] = label


def _stage1(cls3d):
    B = cls3d.shape[0]
    return pl.pallas_call(
        _scores_body,
        grid=(B,),
        in_specs=[pl.BlockSpec((1, 3, NPAD), lambda b: (b, 0, 0))],
        out_specs=[pl.BlockSpec((1, 1, NPAD), lambda b: (b, 0, 0)),
                   pl.BlockSpec((1, 1, NPAD), lambda b: (b, 0, 0))],
        out_shape=[jax.ShapeDtypeStruct((B, 1, NPAD), jnp.float32),
                   jax.ShapeDtypeStruct((B, 1, NPAD), jnp.int32)],
    )(cls3d)


def _rank_body(keyc_ref, keyr_ref, rank_ref):
    ib = pl.program_id(1)
    ki = keyc_ref[0]
    ii = ib * TB + jax.lax.broadcasted_iota(jnp.int32, (TB, 1), 0)

    def chunk(c, acc):
        kj = keyr_ref[0, :, pl.ds(c * RCHUNK, RCHUNK)]
        jj = c * RCHUNK + jax.lax.broadcasted_iota(jnp.int32, (1, RCHUNK), 1)
        cmp = (kj > ki) | ((kj == ki) & (jj < ii))
        return acc + jnp.sum(cmp.astype(jnp.int32), axis=1, keepdims=True)

    acc = jax.lax.fori_loop(0, NPAD // RCHUNK, chunk,
                            jnp.zeros((TB, 1), jnp.int32))
    rank_ref[0] = acc


def _stage2(key):
    B = key.shape[0]
    keyc = key.reshape(B, NPAD, 1)
    keyr = key.reshape(B, 1, NPAD)
    out = pl.pallas_call(
        _rank_body,
        grid=(B, NPAD // TB),
        in_specs=[pl.BlockSpec((1, TB, 1), lambda b, ib: (b, ib, 0)),
                  pl.BlockSpec((1, 1, NPAD), lambda b, ib: (b, 0, 0))],
        out_specs=pl.BlockSpec((1, TB, 1), lambda b, ib: (b, ib, 0)),
        out_shape=jax.ShapeDtypeStruct((B, NPAD, 1), jnp.int32),
    )(keyc, keyr)
    return out.reshape(B, NPAD)


def _nms_body(icols_ref, tblT_ref, keep_ref, alive_s, mloc_s):
    kb = pl.program_id(1)
    i0 = kb * TB
    lane1 = jax.lax.broadcasted_iota(jnp.int32, (1, NCHUNK), 1)
    laneT = jax.lax.broadcasted_iota(jnp.int32, (1, TB), 1)

    @pl.when(kb == 0)
    def _init():
        alive_s[0:1, :] = (tblT_ref[0, 7:8, :] > 0.0).astype(jnp.float32)

    xi = icols_ref[0, :, 0:1]
    yi = icols_ref[0, :, 1:2]
    zi = icols_ref[0, :, 2:3]
    hxi = jnp.abs(icols_ref[0, :, 3:4]) * 0.5
    hyi = jnp.abs(icols_ref[0, :, 4:5]) * 0.5
    hzi = jnp.abs(icols_ref[0, :, 5:6]) * 0.5
    voli = hxi * hyi * hzi * 8.0

    def iou_tile(xj, yj, zj, hxj, hyj, hzj, volj):
        ix = jnp.maximum(jnp.minimum(xi + hxi, xj + hxj) - jnp.maximum(xi - hxi, xj - hxj), 0.0)
        iy = jnp.maximum(jnp.minimum(yi + hyi, yj + hyj) - jnp.maximum(yi - hyi, yj - hyj), 0.0)
        iz = jnp.maximum(jnp.minimum(zi + hzi, zj + hzj) - jnp.maximum(zi - hzi, zj - hzj), 0.0)
        inter = ix * iy * iz
        return inter / (voli + volj - inter + 1e-8)

    def planes(j0, w):
        xj = tblT_ref[0, 0:1, pl.ds(j0, w)]
        yj = tblT_ref[0, 1:2, pl.ds(j0, w)]
        zj = tblT_ref[0, 2:3, pl.ds(j0, w)]
        hxj = jnp.abs(tblT_ref[0, 3:4, pl.ds(j0, w)]) * 0.5
        hyj = jnp.abs(tblT_ref[0, 4:5, pl.ds(j0, w)]) * 0.5
        hzj = jnp.abs(tblT_ref[0, 5:6, pl.ds(j0, w)]) * 0.5
        return xj, yj, zj, hxj, hyj, hzj, hxj * hyj * hzj * 8.0

    mloc_s[:, :] = (iou_tile(*planes(i0, TB)) > NMS_THRESH).astype(jnp.float32)
    al = alive_s[0:1, pl.ds(i0, TB)]

    def scan_i(i, al):
        row = mloc_s[pl.ds(i, 1), :]
        ai = jnp.sum(jnp.where(laneT == i, al, 0.0))
        sup = (row > 0.5) & (laneT > i) & (ai > 0.0)
        return jnp.where(sup, 0.0, al)

    al = jax.lax.fori_loop(0, TB, scan_i, al)
    alive_s[0:1, pl.ds(i0, TB)] = al

    def chunk(c, _):
        j0 = c * NCHUNK

        @pl.when(j0 + NCHUNK > i0 + TB)
        def _():
            Mc = (iou_tile(*planes(j0, NCHUNK)) > NMS_THRESH).astype(jnp.float32)
            sup = jnp.dot(al, Mc, preferred_element_type=jnp.float32)
            later = (j0 + lane1) >= i0 + TB
            hit = (sup > 0.5) & later
            cur = alive_s[0:1, pl.ds(j0, NCHUNK)]
            alive_s[0:1, pl.ds(j0, NCHUNK)] = jnp.where(hit, 0.0, cur)
        return 0

    jax.lax.fori_loop(0, K // NCHUNK, chunk, 0)
    keep_ref[0, 0:1, :] = alive_s[0:1, :]


def _stage4(icols, tblT):
    B = icols.shape[0]
    return pl.pallas_call(
        _nms_body,
        grid=(B, K // TB),
        in_specs=[pl.BlockSpec((1, TB, 8), lambda b, kb: (b, kb, 0)),
                  pl.BlockSpec((1, 8, K), lambda b, kb: (b, 0, 0))],
        out_specs=pl.BlockSpec((1, 1, K), lambda b, kb: (b, 0, 0)),
        out_shape=jax.ShapeDtypeStruct((B, 1, K), jnp.float32),
        scratch_shapes=[pltpu.VMEM((1, K), jnp.float32),
                        pltpu.VMEM((TB, TB), jnp.float32)],
    )(icols, tblT)


_NC, _NS, _NW = 2, 16, 32
_CH = NPAD // _NW
_SPAD = K + 256
_B = 4

_sc_mesh = plsc.VectorSubcoreMesh(
    core_axis_name="c", subcore_axis_name="s", num_cores=_NC, num_subcores=_NS)


def _sc_scatter_body(rank_hbm, table_hbm, sorted_hbm, rank_v, rows_v, idx2d, sem):
    wid = lax.axis_index("s") * _NC + lax.axis_index("c")
    for b in range(_B):
        base = b * NPAD + wid * _CH
        pltpu.sync_copy(rank_hbm.at[pl.ds(base, _CH)], rank_v)
        pltpu.sync_copy(table_hbm.at[pl.ds(base, _CH), :], rows_v)
        for j in range(_CH // 128):
            for i in range(8):
                r = rank_v[pl.ds(j * 128 + i * 16, 16)]
                idx2d[j, pl.ds(i * 16, 16)] = jnp.where(r < K, r, K) + b * _SPAD
        cps = [pltpu.async_copy(rows_v.at[pl.ds(j * 128, 128), :],
                                sorted_hbm.at[idx2d.at[j]], sem)
               for j in range(_CH // 128)]
        for cp in cps:
            cp.wait()


def _stage3(rank_flat, table_flat):
    return pl.kernel(
        _sc_scatter_body,
        out_type=jax.ShapeDtypeStruct((_B * _SPAD, 16), jnp.float32),
        mesh=_sc_mesh,
        compiler_params=pltpu.CompilerParams(use_tc_tiling_on_sc=False, needs_layout_passes=False),
        scratch_types=[pltpu.VMEM((_CH,), jnp.int32),
                       pltpu.VMEM((_CH, 16), jnp.float32),
                       pltpu.VMEM((_CH // 128, 128), jnp.int32),
                       pltpu.SemaphoreType.DMA],
    )(rank_flat, table_flat)


_OUTP = 512


def _sc_select_body(keep_hbm, sorted_hbm, fb_hbm, fs_hbm, fl_hbm,
                    keep_v, sel_v, idx2d, rows_v, sc_v, lb_v, sem):
    wid = lax.axis_index("s") * _NC + lax.axis_index("c")
    b = wid

    @pl.when(wid < _B)
    def _():
        pltpu.sync_copy(keep_hbm.at[pl.ds(b * K, K)], keep_v)

        def comp(i, cnt):
            kv = keep_v[pl.ds(i * 16, 16)]
            mask = kv > 0.5
            pref = plsc.cumsum(jnp.where(mask, 1, 0))
            posv = i * 16 + lax.iota(jnp.int32, 16)
            dst = jnp.where(mask, cnt + pref - 1, K + 8)
            plsc.store_scatter(sel_v, [dst], posv)
            return cnt + jnp.max(pref)

        count = lax.fori_loop(0, K // 16, comp, 0)

        for i in range(_OUTP // 16):
            sl = sel_v[pl.ds(i * 16, 16)]
            slot = i * 16 + lax.iota(jnp.int32, 16)
            idx = jnp.where(slot < count, sl, K) + b * _SPAD
            idx2d[i // 8, pl.ds((i % 8) * 16, 16)] = idx

        cps = [pltpu.async_copy(sorted_hbm.at[idx2d.at[j]],
                                rows_v.at[pl.ds(j * 128, 128), :], sem)
               for j in range(_OUTP // 128)]
        for cp in cps:
            cp.wait()

        def zero(i, _):
            vf = jnp.where(i < count, 1.0, 0.0)
            rows_v[i] = rows_v[i] * vf
            return 0

        lax.fori_loop(0, _OUTP, zero, 0)

        def cols(i, _):
            slot = i * 16 + lax.iota(jnp.int32, 16)
            sc = plsc.load_gather(rows_v, [slot, jnp.full((16,), 7, jnp.int32)])
            lb = plsc.load_gather(rows_v, [slot, jnp.full((16,), 8, jnp.int32)])
            sc_v[pl.ds(i * 16, 16)] = sc
            lb_v[pl.ds(i * 16, 16)] = lb.astype(jnp.int32)
            return 0

        lax.fori_loop(0, _OUTP // 16, cols, 0)

        pltpu.sync_copy(rows_v, fb_hbm.at[pl.ds(b * _OUTP, _OUTP), :])
        pltpu.sync_copy(sc_v, fs_hbm.at[pl.ds(b * _OUTP, _OUTP)])
        pltpu.sync_copy(lb_v, fl_hbm.at[pl.ds(b * _OUTP, _OUTP)])


def _stage5(keep_flat, sorted_flat):
    return pl.kernel(
        _sc_select_body,
        out_type=[jax.ShapeDtypeStruct((_B * _OUTP, 16), jnp.float32),
                  jax.ShapeDtypeStruct((_B * _OUTP,), jnp.float32),
                  jax.ShapeDtypeStruct((_B * _OUTP,), jnp.int32)],
        mesh=_sc_mesh,
        compiler_params=pltpu.CompilerParams(use_tc_tiling_on_sc=False, needs_layout_passes=False),
        scratch_types=[pltpu.VMEM((K,), jnp.float32),
                       pltpu.VMEM((K + 16,), jnp.int32),
                       pltpu.VMEM((_OUTP // 128, 128), jnp.int32),
                       pltpu.VMEM((_OUTP, 16), jnp.float32),
                       pltpu.VMEM((_OUTP,), jnp.float32),
                       pltpu.VMEM((_OUTP,), jnp.int32),
                       pltpu.SemaphoreType.DMA],
    )(keep_flat, sorted_flat)


@jax.jit
def _run(batch_box_preds, batch_cls_preds):
    B, n, _ = batch_box_preds.shape
    cls3d = jnp.pad(batch_cls_preds, ((0, 0), (0, NPAD - n), (0, 0)),
                    constant_values=-30.0).transpose(0, 2, 1)
    boxes = jnp.pad(batch_box_preds, ((0, 0), (0, NPAD - n), (0, 0)))

    masked3, label3 = _stage1(cls3d)
    masked = masked3.reshape(B, NPAD)
    label = label3.reshape(B, NPAD)
    key = jax.lax.bitcast_convert_type(masked, jnp.int32)
    rank = _stage2(key)

    table = jnp.concatenate(
        [boxes, masked[..., None], label.astype(jnp.float32)[..., None],
         jnp.zeros((B, NPAD, 7), jnp.float32)], axis=-1)

    sorted_flat = _stage3(rank.reshape(B * NPAD), table.reshape(B * NPAD, 16))
    sorted4 = sorted_flat.reshape(B, _SPAD, 16)
    icols = sorted4[:, :K, :8]
    tblT = icols.transpose(0, 2, 1)

    keep3 = _stage4(icols, tblT)

    fbf, fsf, flf = _stage5(keep3.reshape(B * K), sorted_flat)
    fb = fbf.reshape(B, _OUTP, 16)[:, :OUT, :7]
    fs = fsf.reshape(B, _OUTP)[:, :OUT]
    fl = flf.reshape(B, _OUTP)[:, :OUT]
    return fb, fs, fl


def kernel(batch_box_preds, batch_cls_preds):
    return _run(batch_box_preds, batch_cls_preds)


# speedup vs baseline: 131.5282x; 4.6930x over previous
import functools
import jax
import jax.numpy as jnp
from jax import lax
from jax.experimental import pallas as pl
from jax.experimental.pallas import tpu as pltpu
from jax.experimental.pallas import tpu_sc as plsc

SCORE_THRESH = 0.1
NMS_THRESH = 0.01
K = 4096
OUT = 500
NPAD = 20480
TB = 256
RCHUNK = 2048
NCHUNK = 1024

PSTRIDE = 20
NPIV = NPAD // PSTRIDE
S_SEL = 304
C = 8192


def _scores_body(cls_ref, masked_ref, label_ref):
    c0 = cls_ref[0, 0:1, :]
    c1 = cls_ref[0, 1:2, :]
    c2 = cls_ref[0, 2:3, :]
    s0 = jax.nn.sigmoid(c0)
    s1 = jax.nn.sigmoid(c1)
    s2 = jax.nn.sigmoid(c2)
    m01 = jnp.maximum(s0, s1)
    i01 = jnp.where(s1 > s0, 1, 0)
    score = jnp.maximum(m01, s2)
    label = jnp.where(s2 > m01, 2, i01) + 1
    masked = jnp.where(score >= SCORE_THRESH, score, -1.0)
    masked_ref[0, :, :] = masked
    label_ref[0, :, :] = label


def _stage1(cls3d):
    B = cls3d.shape[0]
    return pl.pallas_call(
        _scores_body,
        grid=(B,),
        in_specs=[pl.BlockSpec((1, 3, NPAD), lambda b: (b, 0, 0))],
        out_specs=[pl.BlockSpec((1, 1, NPAD), lambda b: (b, 0, 0)),
                   pl.BlockSpec((1, 1, NPAD), lambda b: (b, 0, 0))],
        out_shape=[jax.ShapeDtypeStruct((B, 1, NPAD), jnp.float32),
                   jax.ShapeDtypeStruct((B, 1, NPAD), jnp.int32)],
    )(cls3d)


def _pivrank_body(keyc_ref, keyr_ref, rank_ref):
    ib = pl.program_id(1)
    ki = keyc_ref[0]
    ii = ib * TB + jax.lax.broadcasted_iota(jnp.int32, (TB, 1), 0)
    kj = keyr_ref[0]
    jj = jax.lax.broadcasted_iota(jnp.int32, (1, NPIV), 1)
    cmp = (kj > ki) | ((kj == ki) & (jj < ii))
    rank_ref[0] = jnp.sum(cmp.astype(jnp.int32), axis=1, keepdims=True)


def _stage2a(piv):
    B = piv.shape[0]
    out = pl.pallas_call(
        _pivrank_body,
        grid=(B, NPIV // TB),
        in_specs=[pl.BlockSpec((1, TB, 1), lambda b, ib: (b, ib, 0)),
                  pl.BlockSpec((1, 1, NPIV), lambda b, ib: (b, 0, 0))],
        out_specs=pl.BlockSpec((1, TB, 1), lambda b, ib: (b, ib, 0)),
        out_shape=jax.ShapeDtypeStruct((B, NPIV, 1), jnp.int32),
    )(piv.reshape(B, NPIV, 1), piv.reshape(B, 1, NPIV))
    return out.reshape(B, NPIV)


def _arank_body(keyc_ref, idxc_ref, keyr_ref, idxr_ref, acnt_ref, rank_ref):
    ib = pl.program_id(1)
    ki = keyc_ref[0]
    ii = idxc_ref[0]
    islot = ib * TB + jax.lax.broadcasted_iota(jnp.int32, (TB, 1), 0)
    acv = jnp.max(acnt_ref[0])

    def chunk(c, acc):
        kj = keyr_ref[0, :, pl.ds(c * RCHUNK, RCHUNK)]
        jj = idxr_ref[0, :, pl.ds(c * RCHUNK, RCHUNK)]
        jslot = c * RCHUNK + jax.lax.broadcasted_iota(jnp.int32, (1, RCHUNK), 1)
        cmp = ((kj > ki) | ((kj == ki) & (jj < ii))) & (jslot < acv)
        return acc + jnp.sum(cmp.astype(jnp.int32), axis=1, keepdims=True)

    acc = jax.lax.fori_loop(0, C // RCHUNK, chunk,
                            jnp.zeros((TB, 1), jnp.int32))
    rank_ref[0] = jnp.where(islot < acv, acc, K + islot)


def _stage2b(akey, aidx, acnt):
    B = akey.shape[0]
    out = pl.pallas_call(
        _arank_body,
        grid=(B, C // TB),
        in_specs=[pl.BlockSpec((1, TB, 1), lambda b, ib: (b, ib, 0)),
                  pl.BlockSpec((1, TB, 1), lambda b, ib: (b, ib, 0)),
                  pl.BlockSpec((1, 1, C), lambda b, ib: (b, 0, 0)),
                  pl.BlockSpec((1, 1, C), lambda b, ib: (b, 0, 0)),
                  pl.BlockSpec((1, 1, 16), lambda b, ib: (b, 0, 0))],
        out_specs=pl.BlockSpec((1, TB, 1), lambda b, ib: (b, ib, 0)),
        out_shape=jax.ShapeDtypeStruct((B, C, 1), jnp.int32),
    )(akey.reshape(B, C, 1), aidx.reshape(B, C, 1),
      akey.reshape(B, 1, C), aidx.reshape(B, 1, C), acnt.reshape(B, 1, 16))
    return out.reshape(B, C)


def _nms_body(icols_ref, tblT_ref, keep_ref, alive_s):
    kb = pl.program_id(1)
    i0 = kb * TB
    lane1 = jax.lax.broadcasted_iota(jnp.int32, (1, NCHUNK), 1)
    laneT = jax.lax.broadcasted_iota(jnp.int32, (1, TB), 1)

    @pl.when(kb == 0)
    def _init():
        alive_s[0:1, :] = (tblT_ref[0, 7:8, :] > 0.0).astype(jnp.float32)

    xi = icols_ref[0, :, 0:1]
    yi = icols_ref[0, :, 1:2]
    zi = icols_ref[0, :, 2:3]
    hxi = jnp.abs(icols_ref[0, :, 3:4]) * 0.5
    hyi = jnp.abs(icols_ref[0, :, 4:5]) * 0.5
    hzi = jnp.abs(icols_ref[0, :, 5:6]) * 0.5
    voli = hxi * hyi * hzi * 8.0

    def iou_tile(xj, yj, zj, hxj, hyj, hzj, volj):
        ix = jnp.maximum(jnp.minimum(xi + hxi, xj + hxj) - jnp.maximum(xi - hxi, xj - hxj), 0.0)
        iy = jnp.maximum(jnp.minimum(yi + hyi, yj + hyj) - jnp.maximum(yi - hyi, yj - hyj), 0.0)
        iz = jnp.maximum(jnp.minimum(zi + hzi, zj + hzj) - jnp.maximum(zi - hzi, zj - hzj), 0.0)
        inter = ix * iy * iz
        return inter / (voli + volj - inter + 1e-8)

    def planes(j0, w):
        xj = tblT_ref[0, 0:1, pl.ds(j0, w)]
        yj = tblT_ref[0, 1:2, pl.ds(j0, w)]
        zj = tblT_ref[0, 2:3, pl.ds(j0, w)]
        hxj = jnp.abs(tblT_ref[0, 3:4, pl.ds(j0, w)]) * 0.5
        hyj = jnp.abs(tblT_ref[0, 4:5, pl.ds(j0, w)]) * 0.5
        hzj = jnp.abs(tblT_ref[0, 5:6, pl.ds(j0, w)]) * 0.5
        return xj, yj, zj, hxj, hyj, hzj, hxj * hyj * hzj * 8.0

    rowT = jax.lax.broadcasted_iota(jnp.int32, (TB, TB), 0)
    colT = jax.lax.broadcasted_iota(jnp.int32, (TB, TB), 1)
    mup = jnp.where((iou_tile(*planes(i0, TB)) > NMS_THRESH) & (colT > rowT),
                    1.0, 0.0)
    valid_loc = alive_s[0:1, pl.ds(i0, TB)]

    def jac_cond(c):
        _, changed, it = c
        return changed & (it < TB + 2)

    def jac_body(c):
        al, _, it = c
        sup = jnp.dot(al, mup, preferred_element_type=jnp.float32)
        new = jnp.where(sup > 0.5, 0.0, valid_loc)
        return new, jnp.sum(jnp.abs(new - al)) > 0.0, it + 1

    al, _, _ = jax.lax.while_loop(
        jac_cond, jac_body, (valid_loc, jnp.bool_(True), jnp.int32(0)))
    alive_s[0:1, pl.ds(i0, TB)] = al

    def chunk(c, _):
        j0 = c * NCHUNK

        @pl.when(j0 + NCHUNK > i0 + TB)
        def _():
            Mc = (iou_tile(*planes(j0, NCHUNK)) > NMS_THRESH).astype(jnp.float32)
            sup = jnp.dot(al, Mc, preferred_element_type=jnp.float32)
            later = (j0 + lane1) >= i0 + TB
            hit = (sup > 0.5) & later
            cur = alive_s[0:1, pl.ds(j0, NCHUNK)]
            alive_s[0:1, pl.ds(j0, NCHUNK)] = jnp.where(hit, 0.0, cur)
        return 0

    jax.lax.fori_loop(0, K // NCHUNK, chunk, 0)
    keep_ref[0, 0:1, :] = alive_s[0:1, :]


def _stage4(icols, tblT):
    B = icols.shape[0]
    return pl.pallas_call(
        _nms_body,
        grid=(B, K // TB),
        in_specs=[pl.BlockSpec((1, TB, 8), lambda b, kb: (b, kb, 0)),
                  pl.BlockSpec((1, 8, K), lambda b, kb: (b, 0, 0))],
        out_specs=pl.BlockSpec((1, 1, K), lambda b, kb: (b, 0, 0)),
        out_shape=jax.ShapeDtypeStruct((B, 1, K), jnp.float32),
        scratch_shapes=[pltpu.VMEM((1, K), jnp.float32)],
    )(icols, tblT)


_NC, _NS, _NW = 2, 16, 32
_SPAD = K + C
_B = 4
_ATR = 2 * NPAD

_sc_mesh = plsc.VectorSubcoreMesh(
    core_axis_name="c", subcore_axis_name="s", num_cores=_NC, num_subcores=_NS)

_SC_PARAMS = pltpu.CompilerParams(use_tc_tiling_on_sc=False,
                                  needs_layout_passes=False)


_CHA = NPAD // _NS


def _sc_compact_body(key_hbm, table_hbm, tk_hbm, ti_hbm, atbl_hbm, acnt_hbm,
                     keys_v, mask_v, rows_v, idx2d, tk_v, ti_v, acv_v,
                     ctr_smem, sem):
    cid = lax.axis_index("c")
    sid = lax.axis_index("s")

    @pl.when(sid == 0)
    def _():
        ctr_smem[0] = 0
        ctr_smem[1] = 0

    plsc.subcore_barrier()

    for b2 in range(2):
        b = cid * 2 + b2
        base = b * NPAD + sid * _CHA
        pltpu.sync_copy(key_hbm.at[pl.ds(base, _CHA)], keys_v)
        pltpu.sync_copy(tk_hbm.at[b], tk_v)
        pltpu.sync_copy(ti_hbm.at[b], ti_v)
        tk = jnp.max(tk_v[...])
        ti = jnp.max(ti_v[...])

        acc = jnp.zeros((16,), jnp.int32)
        for v in range(_CHA // 16):
            k16 = keys_v[pl.ds(v * 16, 16)]
            gi = sid * _CHA + v * 16 + lax.iota(jnp.int32, 16)
            m = (k16 > tk) | ((k16 == tk) & (gi <= ti))
            mask_v[pl.ds(v * 16, 16)] = jnp.where(m, 1, 0)
            acc = acc + jnp.where(m, 1, 0)
        cnt = jnp.sum(acc)

        base_slot = plsc.fetch_and_add(ctr_smem.at[b2], cnt, subcore_id=0)

        run = base_slot
        for v in range(_CHA // 16):
            mv = mask_v[pl.ds(v * 16, 16)]
            pref = plsc.cumsum(mv)
            slotg = sid * _CHA + v * 16 + lax.iota(jnp.int32, 16)
            dst = jnp.where(mv > 0, run + pref - 1, NPAD + slotg) + b * _ATR
            idx2d[v // 8, pl.ds((v % 8) * 16, 16)] = dst
            run = run + jnp.max(pref)

        pltpu.sync_copy(table_hbm.at[pl.ds(base, _CHA), :], rows_v)
        cps = [pltpu.async_copy(rows_v.at[pl.ds(j * 128, 128), :],
                                atbl_hbm.at[idx2d.at[j]], sem)
               for j in range(_CHA // 128)]
        for cp in cps:
            cp.wait()

    plsc.subcore_barrier()

    @pl.when(sid == 0)
    def _():
        for b2 in range(2):
            b = cid * 2 + b2
            acv_v[...] = jnp.zeros((16,), jnp.int32) + ctr_smem[b2]
            pltpu.sync_copy(acv_v, acnt_hbm.at[b])


def _stageA(key_flat, table_flat, tk16, ti16):
    return pl.kernel(
        _sc_compact_body,
        out_type=[jax.ShapeDtypeStruct((_B * _ATR, 16), jnp.float32),
                  jax.ShapeDtypeStruct((_B, 16), jnp.int32)],
        mesh=_sc_mesh,
        compiler_params=_SC_PARAMS,
        scratch_types=[pltpu.VMEM((_CHA,), jnp.int32),
                       pltpu.VMEM((_CHA,), jnp.int32),
                       pltpu.VMEM((_CHA, 16), jnp.float32),
                       pltpu.VMEM((_CHA // 128, 128), jnp.int32),
                       pltpu.VMEM((16,), jnp.int32),
                       pltpu.VMEM((16,), jnp.int32),
                       pltpu.VMEM((16,), jnp.int32),
                       pltpu.SMEM((8,), jnp.int32),
                       pltpu.SemaphoreType.DMA],
    )(key_flat, table_flat, tk16, ti16)


_CH3 = C // _NS


def _sc_scatter_body(rank_hbm, atbl_hbm, sorted_hbm, rank_v, rows_v, idx2d, sem):
    cid = lax.axis_index("c")
    sid = lax.axis_index("s")
    for b2 in range(2):
        b = cid * 2 + b2
        rbase = b * C + sid * _CH3
        tbase = b * _ATR + sid * _CH3
        pltpu.sync_copy(rank_hbm.at[pl.ds(rbase, _CH3)], rank_v)
        pltpu.sync_copy(atbl_hbm.at[pl.ds(tbase, _CH3), :], rows_v)
        for j in range(_CH3 // 128):
            for i in range(8):
                off = j * 128 + i * 16
                r = rank_v[pl.ds(off, 16)]
                trash = K + sid * _CH3 + off + lax.iota(jnp.int32, 16)
                idx2d[j, pl.ds(i * 16, 16)] = jnp.where(r < K, r, trash) + b * _SPAD
        cps = [pltpu.async_copy(rows_v.at[pl.ds(j * 128, 128), :],
                                sorted_hbm.at[idx2d.at[j]], sem)
               for j in range(_CH3 // 128)]
        for cp in cps:
            cp.wait()


def _stage3(rank_flat, atbl_flat):
    return pl.kernel(
        _sc_scatter_body,
        out_type=jax.ShapeDtypeStruct((_B * _SPAD, 16), jnp.float32),
        mesh=_sc_mesh,
        compiler_params=_SC_PARAMS,
        scratch_types=[pltpu.VMEM((_CH3,), jnp.int32),
                       pltpu.VMEM((_CH3, 16), jnp.float32),
                       pltpu.VMEM((_CH3 // 128, 128), jnp.int32),
                       pltpu.SemaphoreType.DMA],
    )(rank_flat, atbl_flat)


_OUTP = 512


def _sc_select_body(keep_hbm, sorted_hbm, fb_hbm, fs_hbm, fl_hbm,
                    keep_v, sel_v, idx2d, rows_v, sc_v, lb_v, sem):
    wid = lax.axis_index("s") * _NC + lax.axis_index("c")
    b = wid

    @pl.when(wid < _B)
    def _():
        pltpu.sync_copy(keep_hbm.at[pl.ds(b * K, K)], keep_v)

        def comp(i, cnt):
            kv = keep_v[pl.ds(i * 16, 16)]
            mask = kv > 0.5
            pref = plsc.cumsum(jnp.where(mask, 1, 0))
            posv = i * 16 + lax.iota(jnp.int32, 16)
            dst = jnp.where(mask, cnt + pref - 1, K + 8)
            plsc.store_scatter(sel_v, [dst], posv)
            return cnt + jnp.max(pref)

        count = lax.fori_loop(0, K // 16, comp, 0)

        for i in range(_OUTP // 16):
            sl = sel_v[pl.ds(i * 16, 16)]
            slot = i * 16 + lax.iota(jnp.int32, 16)
            idx = jnp.where(slot < count, sl, 0) + b * _SPAD
            idx2d[i // 8, pl.ds((i % 8) * 16, 16)] = idx

        cps = [pltpu.async_copy(sorted_hbm.at[idx2d.at[j]],
                                rows_v.at[pl.ds(j * 128, 128), :], sem)
               for j in range(_OUTP // 128)]
        for cp in cps:
            cp.wait()

        def zero(i, _):
            vf = jnp.where(i < count, 1.0, 0.0)
            rows_v[i] = rows_v[i] * vf
            return 0

        lax.fori_loop(0, _OUTP, zero, 0)

        def cols(i, _):
            slot = i * 16 + lax.iota(jnp.int32, 16)
            sc = plsc.load_gather(rows_v, [slot, jnp.full((16,), 7, jnp.int32)])
            lb = plsc.load_gather(rows_v, [slot, jnp.full((16,), 8, jnp.int32)])
            sc_v[pl.ds(i * 16, 16)] = sc
            lb_v[pl.ds(i * 16, 16)] = lb.astype(jnp.int32)
            return 0

        lax.fori_loop(0, _OUTP // 16, cols, 0)

        pltpu.sync_copy(rows_v, fb_hbm.at[pl.ds(b * _OUTP, _OUTP), :])
        pltpu.sync_copy(sc_v, fs_hbm.at[pl.ds(b * _OUTP, _OUTP)])
        pltpu.sync_copy(lb_v, fl_hbm.at[pl.ds(b * _OUTP, _OUTP)])


def _stage5(keep_flat, sorted_flat):
    return pl.kernel(
        _sc_select_body,
        out_type=[jax.ShapeDtypeStruct((_B * _OUTP, 16), jnp.float32),
                  jax.ShapeDtypeStruct((_B * _OUTP,), jnp.float32),
                  jax.ShapeDtypeStruct((_B * _OUTP,), jnp.int32)],
        mesh=_sc_mesh,
        compiler_params=pltpu.CompilerParams(use_tc_tiling_on_sc=False, needs_layout_passes=False),
        scratch_types=[pltpu.VMEM((K,), jnp.float32),
                       pltpu.VMEM((K + 16,), jnp.int32),
                       pltpu.VMEM((_OUTP // 128, 128), jnp.int32),
                       pltpu.VMEM((_OUTP, 16), jnp.float32),
                       pltpu.VMEM((_OUTP,), jnp.float32),
                       pltpu.VMEM((_OUTP,), jnp.int32),
                       pltpu.SemaphoreType.DMA],
    )(keep_flat, sorted_flat)


@jax.jit
def _run(batch_box_preds, batch_cls_preds):
    B, n, _ = batch_box_preds.shape
    cls3d = jnp.pad(batch_cls_preds, ((0, 0), (0, NPAD - n), (0, 0)),
                    constant_values=-30.0).transpose(0, 2, 1)
    boxes = jnp.pad(batch_box_preds, ((0, 0), (0, NPAD - n), (0, 0)))

    masked3, label3 = _stage1(cls3d)
    masked = masked3.reshape(B, NPAD)
    label = label3.reshape(B, NPAD)
    key = jax.lax.bitcast_convert_type(masked, jnp.int32)

    piv = key[:, ::PSTRIDE]
    rankp = _stage2a(piv)
    sel = rankp == S_SEL
    tkey = jnp.sum(jnp.where(sel, piv, 0), axis=1)
    tm = jnp.sum(jnp.where(sel, jnp.arange(NPIV, dtype=jnp.int32)[None, :], 0),
                 axis=1)
    tk16 = jnp.broadcast_to(tkey[:, None], (B, 16)).astype(jnp.int32)
    ti16 = jnp.broadcast_to((tm * PSTRIDE)[:, None], (B, 16)).astype(jnp.int32)

    oidx = jnp.broadcast_to(
        jnp.arange(NPAD, dtype=jnp.float32)[None, :, None], (B, NPAD, 1))
    table = jnp.concatenate(
        [boxes, masked[..., None], label.astype(jnp.float32)[..., None],
         oidx, jnp.zeros((B, NPAD, 6), jnp.float32)], axis=-1)

    atbl_flat, acnt = _stageA(key.reshape(B * NPAD),
                              table.reshape(B * NPAD, 16), tk16, ti16)
    atbl4 = atbl_flat.reshape(B, _ATR, 16)
    akey = jax.lax.bitcast_convert_type(atbl4[:, :C, 7], jnp.int32)
    aidx = atbl4[:, :C, 9].astype(jnp.int32)
    rankA = _stage2b(akey, aidx, acnt)

    sorted_flat = _stage3(rankA.reshape(B * C), atbl_flat)
    sorted4 = sorted_flat.reshape(B, _SPAD, 16)
    icols = sorted4[:, :K, :8]
    tblT = icols.transpose(0, 2, 1)

    keep3 = _stage4(icols, tblT)

    fbf, fsf, flf = _stage5(keep3.reshape(B * K), sorted_flat)
    fb = fbf.reshape(B, _OUTP, 16)[:, :OUT, :7]
    fs = fsf.reshape(B, _OUTP)[:, :OUT]
    fl = flf.reshape(B, _OUTP)[:, :OUT]
    return fb, fs, fl


def kernel(batch_box_preds, batch_cls_preds):
    return _run(batch_box_preds, batch_cls_preds)


# speedup vs baseline: 163.3331x; 1.2418x over previous
import functools
import jax
import jax.numpy as jnp
from jax import lax
from jax.experimental import pallas as pl
from jax.experimental.pallas import tpu as pltpu
from jax.experimental.pallas import tpu_sc as plsc

SCORE_THRESH = 0.1
NMS_THRESH = 0.01
K = 4096
OUT = 500
NPAD = 20480
TB = 256
TBN = 512
RCHUNK = 2048
NCHUNK = 1024

PSTRIDE = 20
NPIV = NPAD // PSTRIDE
S_SEL = 304
C = 8192


def _scores_body(cls_ref, masked_ref, label_ref):
    c0 = cls_ref[0, 0:1, :]
    c1 = cls_ref[0, 1:2, :]
    c2 = cls_ref[0, 2:3, :]
    s0 = jax.nn.sigmoid(c0)
    s1 = jax.nn.sigmoid(c1)
    s2 = jax.nn.sigmoid(c2)
    m01 = jnp.maximum(s0, s1)
    i01 = jnp.where(s1 > s0, 1, 0)
    score = jnp.maximum(m01, s2)
    label = jnp.where(s2 > m01, 2, i01) + 1
    masked = jnp.where(score >= SCORE_THRESH, score, -1.0)
    masked_ref[0, :, :] = masked
    label_ref[0, :, :] = label


def _stage1(cls3d):
    B = cls3d.shape[0]
    return pl.pallas_call(
        _scores_body,
        grid=(B,),
        in_specs=[pl.BlockSpec((1, 3, NPAD), lambda b: (b, 0, 0))],
        out_specs=[pl.BlockSpec((1, 1, NPAD), lambda b: (b, 0, 0)),
                   pl.BlockSpec((1, 1, NPAD), lambda b: (b, 0, 0))],
        out_shape=[jax.ShapeDtypeStruct((B, 1, NPAD), jnp.float32),
                   jax.ShapeDtypeStruct((B, 1, NPAD), jnp.int32)],
    )(cls3d)


def _pivrank_body(keyc_ref, keyr_ref, rank_ref):
    ib = pl.program_id(1)
    ki = keyc_ref[0]
    ii = ib * TB + jax.lax.broadcasted_iota(jnp.int32, (TB, 1), 0)
    kj = keyr_ref[0]
    jj = jax.lax.broadcasted_iota(jnp.int32, (1, NPIV), 1)
    cmp = (kj > ki) | ((kj == ki) & (jj < ii))
    rank_ref[0] = jnp.sum(cmp.astype(jnp.int32), axis=1, keepdims=True)


def _stage2a(piv):
    B = piv.shape[0]
    out = pl.pallas_call(
        _pivrank_body,
        grid=(B, NPIV // TB),
        in_specs=[pl.BlockSpec((1, TB, 1), lambda b, ib: (b, ib, 0)),
                  pl.BlockSpec((1, 1, NPIV), lambda b, ib: (b, 0, 0))],
        out_specs=pl.BlockSpec((1, TB, 1), lambda b, ib: (b, ib, 0)),
        out_shape=jax.ShapeDtypeStruct((B, NPIV, 1), jnp.int32),
    )(piv.reshape(B, NPIV, 1), piv.reshape(B, 1, NPIV))
    return out.reshape(B, NPIV)


def _arank_body(keyc_ref, idxc_ref, keyr_ref, idxr_ref, acnt_ref, rank_ref):
    ib = pl.program_id(1)
    ki = keyc_ref[0]
    ii = idxc_ref[0]
    islot = ib * TB + jax.lax.broadcasted_iota(jnp.int32, (TB, 1), 0)
    acv = jnp.max(acnt_ref[0])

    def chunk(c, acc):
        kj = keyr_ref[0, :, pl.ds(c * RCHUNK, RCHUNK)]
        jj = idxr_ref[0, :, pl.ds(c * RCHUNK, RCHUNK)]
        jslot = c * RCHUNK + jax.lax.broadcasted_iota(jnp.int32, (1, RCHUNK), 1)
        cmp = ((kj > ki) | ((kj == ki) & (jj < ii))) & (jslot < acv)
        return acc + jnp.sum(cmp.astype(jnp.int32), axis=1, keepdims=True)

    zero = jnp.zeros((TB, 1), jnp.int32)
    nch = (jnp.minimum(acv, C) + (RCHUNK - 1)) // RCHUNK
    acc = jax.lax.cond(ib * TB < acv,
                       lambda: jax.lax.fori_loop(0, nch, chunk, zero),
                       lambda: zero)
    rank_ref[0] = jnp.where(islot < acv, acc, K + islot)


def _stage2b(akey, aidx, acnt):
    B = akey.shape[0]
    out = pl.pallas_call(
        _arank_body,
        grid=(B, C // TB),
        in_specs=[pl.BlockSpec((1, TB, 1), lambda b, ib: (b, ib, 0)),
                  pl.BlockSpec((1, TB, 1), lambda b, ib: (b, ib, 0)),
                  pl.BlockSpec((1, 1, C), lambda b, ib: (b, 0, 0)),
                  pl.BlockSpec((1, 1, C), lambda b, ib: (b, 0, 0)),
                  pl.BlockSpec((1, 1, 16), lambda b, ib: (b, 0, 0))],
        out_specs=pl.BlockSpec((1, TB, 1), lambda b, ib: (b, ib, 0)),
        out_shape=jax.ShapeDtypeStruct((B, C, 1), jnp.int32),
    )(akey.reshape(B, C, 1), aidx.reshape(B, C, 1),
      akey.reshape(B, 1, C), aidx.reshape(B, 1, C), acnt.reshape(B, 1, 16))
    return out.reshape(B, C)


def _nms_body(icols_ref, tblT_ref, keep_ref, alive_s):
    kb = pl.program_id(1)
    i0 = kb * TBN
    lane1 = jax.lax.broadcasted_iota(jnp.int32, (1, NCHUNK), 1)
    laneT = jax.lax.broadcasted_iota(jnp.int32, (1, TBN), 1)

    @pl.when(kb == 0)
    def _init():
        alive_s[0:1, :] = (tblT_ref[0, 7:8, :] > 0.0).astype(jnp.float32)

    xi = icols_ref[0, :, 0:1]
    yi = icols_ref[0, :, 1:2]
    zi = icols_ref[0, :, 2:3]
    hxi = jnp.abs(icols_ref[0, :, 3:4]) * 0.5
    hyi = jnp.abs(icols_ref[0, :, 4:5]) * 0.5
    hzi = jnp.abs(icols_ref[0, :, 5:6]) * 0.5
    voli = hxi * hyi * hzi * 8.0

    def iou_tile(xj, yj, zj, hxj, hyj, hzj, volj):
        ix = jnp.maximum(jnp.minimum(xi + hxi, xj + hxj) - jnp.maximum(xi - hxi, xj - hxj), 0.0)
        iy = jnp.maximum(jnp.minimum(yi + hyi, yj + hyj) - jnp.maximum(yi - hyi, yj - hyj), 0.0)
        iz = jnp.maximum(jnp.minimum(zi + hzi, zj + hzj) - jnp.maximum(zi - hzi, zj - hzj), 0.0)
        inter = ix * iy * iz
        return inter / (voli + volj - inter + 1e-8)

    def planes(j0, w):
        xj = tblT_ref[0, 0:1, pl.ds(j0, w)]
        yj = tblT_ref[0, 1:2, pl.ds(j0, w)]
        zj = tblT_ref[0, 2:3, pl.ds(j0, w)]
        hxj = jnp.abs(tblT_ref[0, 3:4, pl.ds(j0, w)]) * 0.5
        hyj = jnp.abs(tblT_ref[0, 4:5, pl.ds(j0, w)]) * 0.5
        hzj = jnp.abs(tblT_ref[0, 5:6, pl.ds(j0, w)]) * 0.5
        return xj, yj, zj, hxj, hyj, hzj, hxj * hyj * hzj * 8.0

    rowT = jax.lax.broadcasted_iota(jnp.int32, (TBN, TBN), 0)
    colT = jax.lax.broadcasted_iota(jnp.int32, (TBN, TBN), 1)
    mup = jnp.where((iou_tile(*planes(i0, TBN)) > NMS_THRESH) & (colT > rowT),
                    1.0, 0.0)
    valid_loc = alive_s[0:1, pl.ds(i0, TBN)]

    def jac_cond(c):
        _, changed, it = c
        return changed & (it < TBN + 2)

    def jac_body(c):
        al, _, it = c
        sup = jnp.dot(al, mup, preferred_element_type=jnp.float32)
        new = jnp.where(sup > 0.5, 0.0, valid_loc)
        return new, jnp.sum(jnp.abs(new - al)) > 0.0, it + 1

    al, _, _ = jax.lax.while_loop(
        jac_cond, jac_body, (valid_loc, jnp.bool_(True), jnp.int32(0)))
    alive_s[0:1, pl.ds(i0, TBN)] = al

    def chunk(c, _):
        j0 = c * NCHUNK

        @pl.when(j0 + NCHUNK > i0 + TBN)
        def _():
            Mc = (iou_tile(*planes(j0, NCHUNK)) > NMS_THRESH).astype(jnp.float32)
            sup = jnp.dot(al, Mc, preferred_element_type=jnp.float32)
            later = (j0 + lane1) >= i0 + TBN
            hit = (sup > 0.5) & later
            cur = alive_s[0:1, pl.ds(j0, NCHUNK)]
            alive_s[0:1, pl.ds(j0, NCHUNK)] = jnp.where(hit, 0.0, cur)
        return 0

    jax.lax.fori_loop(0, K // NCHUNK, chunk, 0)
    keep_ref[0, 0:1, :] = alive_s[0:1, :]


def _stage4(icols, tblT):
    B = icols.shape[0]
    return pl.pallas_call(
        _nms_body,
        grid=(B, K // TBN),
        in_specs=[pl.BlockSpec((1, TBN, 8), lambda b, kb: (b, kb, 0)),
                  pl.BlockSpec((1, 8, K), lambda b, kb: (b, 0, 0))],
        out_specs=pl.BlockSpec((1, 1, K), lambda b, kb: (b, 0, 0)),
        out_shape=jax.ShapeDtypeStruct((B, 1, K), jnp.float32),
        scratch_shapes=[pltpu.VMEM((1, K), jnp.float32)],
    )(icols, tblT)


_NC, _NS, _NW = 2, 16, 32
_SPAD = K + C
_B = 4
_ATR = 2 * NPAD

_sc_mesh = plsc.VectorSubcoreMesh(
    core_axis_name="c", subcore_axis_name="s", num_cores=_NC, num_subcores=_NS)

_SC_PARAMS = pltpu.CompilerParams(use_tc_tiling_on_sc=False,
                                  needs_layout_passes=False)


_CHA = NPAD // _NS


def _sc_compact_body(key_hbm, table_hbm, tk_hbm, ti_hbm, atbl_hbm, acnt_hbm,
                     keys_v, mask_v, rows_v, idx2d, tk_v, ti_v, acv_v,
                     ctr_smem, sem):
    cid = lax.axis_index("c")
    sid = lax.axis_index("s")

    @pl.when(sid == 0)
    def _():
        ctr_smem[0] = 0
        ctr_smem[1] = 0

    plsc.subcore_barrier()

    for b2 in range(2):
        b = cid * 2 + b2
        base = b * NPAD + sid * _CHA
        pltpu.sync_copy(key_hbm.at[pl.ds(base, _CHA)], keys_v)
        pltpu.sync_copy(tk_hbm.at[b], tk_v)
        pltpu.sync_copy(ti_hbm.at[b], ti_v)
        tk = jnp.max(tk_v[...])
        ti = jnp.max(ti_v[...])

        acc = jnp.zeros((16,), jnp.int32)
        for v in range(_CHA // 16):
            k16 = keys_v[pl.ds(v * 16, 16)]
            gi = sid * _CHA + v * 16 + lax.iota(jnp.int32, 16)
            m = (k16 > tk) | ((k16 == tk) & (gi <= ti))
            mask_v[pl.ds(v * 16, 16)] = jnp.where(m, 1, 0)
            acc = acc + jnp.where(m, 1, 0)
        cnt = jnp.sum(acc)

        base_slot = plsc.fetch_and_add(ctr_smem.at[b2], cnt, subcore_id=0)

        run = base_slot
        for v in range(_CHA // 16):
            mv = mask_v[pl.ds(v * 16, 16)]
            pref = plsc.cumsum(mv)
            slotg = sid * _CHA + v * 16 + lax.iota(jnp.int32, 16)
            dst = jnp.where(mv > 0, run + pref - 1, NPAD + slotg) + b * _ATR
            idx2d[v // 8, pl.ds((v % 8) * 16, 16)] = dst
            run = run + jnp.max(pref)

        pltpu.sync_copy(table_hbm.at[pl.ds(base, _CHA), :], rows_v)
        cps = [pltpu.async_copy(rows_v.at[pl.ds(j * 128, 128), :],
                                atbl_hbm.at[idx2d.at[j]], sem)
               for j in range(_CHA // 128)]
        for cp in cps:
            cp.wait()

    plsc.subcore_barrier()

    @pl.when(sid == 0)
    def _():
        for b2 in range(2):
            b = cid * 2 + b2
            acv_v[...] = jnp.zeros((16,), jnp.int32) + ctr_smem[b2]
            pltpu.sync_copy(acv_v, acnt_hbm.at[b])


def _stageA(key_flat, table_flat, tk16, ti16):
    return pl.kernel(
        _sc_compact_body,
        out_type=[jax.ShapeDtypeStruct((_B * _ATR, 16), jnp.float32),
                  jax.ShapeDtypeStruct((_B, 16), jnp.int32)],
        mesh=_sc_mesh,
        compiler_params=_SC_PARAMS,
        scratch_types=[pltpu.VMEM((_CHA,), jnp.int32),
                       pltpu.VMEM((_CHA,), jnp.int32),
                       pltpu.VMEM((_CHA, 16), jnp.float32),
                       pltpu.VMEM((_CHA // 128, 128), jnp.int32),
                       pltpu.VMEM((16,), jnp.int32),
                       pltpu.VMEM((16,), jnp.int32),
                       pltpu.VMEM((16,), jnp.int32),
                       pltpu.SMEM((8,), jnp.int32),
                       pltpu.SemaphoreType.DMA],
    )(key_flat, table_flat, tk16, ti16)


_CH3 = C // _NS


def _sc_scatter_body(rank_hbm, atbl_hbm, sorted_hbm, rank_v, rows_v, idx2d, sem):
    cid = lax.axis_index("c")
    sid = lax.axis_index("s")
    for b2 in range(2):
        b = cid * 2 + b2
        rbase = b * C + sid * _CH3
        tbase = b * _ATR + sid * _CH3
        pltpu.sync_copy(rank_hbm.at[pl.ds(rbase, _CH3)], rank_v)
        pltpu.sync_copy(atbl_hbm.at[pl.ds(tbase, _CH3), :], rows_v)
        for j in range(_CH3 // 128):
            for i in range(8):
                off = j * 128 + i * 16
                r = rank_v[pl.ds(off, 16)]
                trash = K + sid * _CH3 + off + lax.iota(jnp.int32, 16)
                idx2d[j, pl.ds(i * 16, 16)] = jnp.where(r < K, r, trash) + b * _SPAD
        cps = [pltpu.async_copy(rows_v.at[pl.ds(j * 128, 128), :],
                                sorted_hbm.at[idx2d.at[j]], sem)
               for j in range(_CH3 // 128)]
        for cp in cps:
            cp.wait()


def _stage3(rank_flat, atbl_flat):
    return pl.kernel(
        _sc_scatter_body,
        out_type=jax.ShapeDtypeStruct((_B * _SPAD, 16), jnp.float32),
        mesh=_sc_mesh,
        compiler_params=_SC_PARAMS,
        scratch_types=[pltpu.VMEM((_CH3,), jnp.int32),
                       pltpu.VMEM((_CH3, 16), jnp.float32),
                       pltpu.VMEM((_CH3 // 128, 128), jnp.int32),
                       pltpu.SemaphoreType.DMA],
    )(rank_flat, atbl_flat)


_OUTP = 512


def _sc_select_body(keep_hbm, sorted_hbm, fb_hbm, fs_hbm, fl_hbm,
                    keep_v, sel_v, idx2d, rows_v, sc_v, lb_v, sem):
    wid = lax.axis_index("s") * _NC + lax.axis_index("c")
    b = wid

    @pl.when(wid < _B)
    def _():
        pltpu.sync_copy(keep_hbm.at[pl.ds(b * K, K)], keep_v)

        def comp(i, cnt):
            kv = keep_v[pl.ds(i * 16, 16)]
            mask = kv > 0.5
            pref = plsc.cumsum(jnp.where(mask, 1, 0))
            posv = i * 16 + lax.iota(jnp.int32, 16)
            dst = jnp.where(mask, cnt + pref - 1, K + 8)
            plsc.store_scatter(sel_v, [dst], posv)
            return cnt + jnp.max(pref)

        count = lax.fori_loop(0, K // 16, comp, 0)

        for i in range(_OUTP // 16):
            sl = sel_v[pl.ds(i * 16, 16)]
            slot = i * 16 + lax.iota(jnp.int32, 16)
            idx = jnp.where(slot < count, sl, 0) + b * _SPAD
            idx2d[i // 8, pl.ds((i % 8) * 16, 16)] = idx

        cps = [pltpu.async_copy(sorted_hbm.at[idx2d.at[j]],
                                rows_v.at[pl.ds(j * 128, 128), :], sem)
               for j in range(_OUTP // 128)]
        for cp in cps:
            cp.wait()

        def zero(i, _):
            vf = jnp.where(i < count, 1.0, 0.0)
            rows_v[i] = rows_v[i] * vf
            return 0

        lax.fori_loop(0, _OUTP, zero, 0)

        def cols(i, _):
            slot = i * 16 + lax.iota(jnp.int32, 16)
            sc = plsc.load_gather(rows_v, [slot, jnp.full((16,), 7, jnp.int32)])
            lb = plsc.load_gather(rows_v, [slot, jnp.full((16,), 8, jnp.int32)])
            sc_v[pl.ds(i * 16, 16)] = sc
            lb_v[pl.ds(i * 16, 16)] = lb.astype(jnp.int32)
            return 0

        lax.fori_loop(0, _OUTP // 16, cols, 0)

        pltpu.sync_copy(rows_v, fb_hbm.at[pl.ds(b * _OUTP, _OUTP), :])
        pltpu.sync_copy(sc_v, fs_hbm.at[pl.ds(b * _OUTP, _OUTP)])
        pltpu.sync_copy(lb_v, fl_hbm.at[pl.ds(b * _OUTP, _OUTP)])


def _stage5(keep_flat, sorted_flat):
    return pl.kernel(
        _sc_select_body,
        out_type=[jax.ShapeDtypeStruct((_B * _OUTP, 16), jnp.float32),
                  jax.ShapeDtypeStruct((_B * _OUTP,), jnp.float32),
                  jax.ShapeDtypeStruct((_B * _OUTP,), jnp.int32)],
        mesh=_sc_mesh,
        compiler_params=pltpu.CompilerParams(use_tc_tiling_on_sc=False, needs_layout_passes=False),
        scratch_types=[pltpu.VMEM((K,), jnp.float32),
                       pltpu.VMEM((K + 16,), jnp.int32),
                       pltpu.VMEM((_OUTP // 128, 128), jnp.int32),
                       pltpu.VMEM((_OUTP, 16), jnp.float32),
                       pltpu.VMEM((_OUTP,), jnp.float32),
                       pltpu.VMEM((_OUTP,), jnp.int32),
                       pltpu.SemaphoreType.DMA],
    )(keep_flat, sorted_flat)


@jax.jit
def _run(batch_box_preds, batch_cls_preds):
    B, n, _ = batch_box_preds.shape
    cls3d = jnp.pad(batch_cls_preds, ((0, 0), (0, NPAD - n), (0, 0)),
                    constant_values=-30.0).transpose(0, 2, 1)
    boxes = jnp.pad(batch_box_preds, ((0, 0), (0, NPAD - n), (0, 0)))

    masked3, label3 = _stage1(cls3d)
    masked = masked3.reshape(B, NPAD)
    label = label3.reshape(B, NPAD)
    key = jax.lax.bitcast_convert_type(masked, jnp.int32)

    piv = key[:, ::PSTRIDE]
    rankp = _stage2a(piv)
    sel = rankp == S_SEL
    tkey = jnp.sum(jnp.where(sel, piv, 0), axis=1)
    tm = jnp.sum(jnp.where(sel, jnp.arange(NPIV, dtype=jnp.int32)[None, :], 0),
                 axis=1)
    tk16 = jnp.broadcast_to(tkey[:, None], (B, 16)).astype(jnp.int32)
    ti16 = jnp.broadcast_to((tm * PSTRIDE)[:, None], (B, 16)).astype(jnp.int32)

    oidx = jnp.broadcast_to(
        jnp.arange(NPAD, dtype=jnp.float32)[None, :, None], (B, NPAD, 1))
    table = jnp.concatenate(
        [boxes, masked[..., None], label.astype(jnp.float32)[..., None],
         oidx, jnp.zeros((B, NPAD, 6), jnp.float32)], axis=-1)

    atbl_flat, acnt = _stageA(key.reshape(B * NPAD),
                              table.reshape(B * NPAD, 16), tk16, ti16)
    atbl4 = atbl_flat.reshape(B, _ATR, 16)
    akey = jax.lax.bitcast_convert_type(atbl4[:, :C, 7], jnp.int32)
    aidx = atbl4[:, :C, 9].astype(jnp.int32)
    rankA = _stage2b(akey, aidx, acnt)

    sorted_flat = _stage3(rankA.reshape(B * C), atbl_flat)
    sorted4 = sorted_flat.reshape(B, _SPAD, 16)
    icols = sorted4[:, :K, :8]
    tblT = icols.transpose(0, 2, 1)

    keep3 = _stage4(icols, tblT)

    fbf, fsf, flf = _stage5(keep3.reshape(B * K), sorted_flat)
    fb = fbf.reshape(B, _OUTP, 16)[:, :OUT, :7]
    fs = fsf.reshape(B, _OUTP)[:, :OUT]
    fl = flf.reshape(B, _OUTP)[:, :OUT]
    return fb, fs, fl


def kernel(batch_box_preds, batch_cls_preds):
    return _run(batch_box_preds, batch_cls_preds)


# speedup vs baseline: 164.3846x; 1.0064x over previous
import jax
import jax.numpy as jnp
from jax import lax
from jax.experimental import pallas as pl
from jax.experimental.pallas import tpu as pltpu
from jax.experimental.pallas import tpu_sc as plsc

SCORE_THRESH = 0.1
NMS_THRESH = 0.01
K = 4096
OUT = 500
NPAD = 20480
TB = 256
TBN = 512
RCHUNK = 2048
NCHUNK = 512

PSTRIDE = 20
NPIV = NPAD // PSTRIDE
S_SEL = 304
C = 8192


def _scores_body(cls_ref, masked_ref, label_ref):
    c0 = cls_ref[0, 0:1, :]
    c1 = cls_ref[0, 1:2, :]
    c2 = cls_ref[0, 2:3, :]
    s0 = jax.nn.sigmoid(c0)
    s1 = jax.nn.sigmoid(c1)
    s2 = jax.nn.sigmoid(c2)
    m01 = jnp.maximum(s0, s1)
    i01 = jnp.where(s1 > s0, 1, 0)
    score = jnp.maximum(m01, s2)
    label = jnp.where(s2 > m01, 2, i01) + 1
    masked = jnp.where(score >= SCORE_THRESH, score, -1.0)
    masked_ref[0, :, :] = masked
    label_ref[0, :, :] = label


def _stage1(cls3d):
    B = cls3d.shape[0]
    return pl.pallas_call(
        _scores_body,
        grid=(B,),
        in_specs=[pl.BlockSpec((1, 3, NPAD), lambda b: (b, 0, 0))],
        out_specs=[pl.BlockSpec((1, 1, NPAD), lambda b: (b, 0, 0)),
                   pl.BlockSpec((1, 1, NPAD), lambda b: (b, 0, 0))],
        out_shape=[jax.ShapeDtypeStruct((B, 1, NPAD), jnp.float32),
                   jax.ShapeDtypeStruct((B, 1, NPAD), jnp.int32)],
    )(cls3d)


def _pivrank_body(keyc_ref, keyr_ref, rank_ref):
    ib = pl.program_id(1)
    ki = keyc_ref[0]
    ii = ib * TB + jax.lax.broadcasted_iota(jnp.int32, (TB, 1), 0)
    kj = keyr_ref[0]
    jj = jax.lax.broadcasted_iota(jnp.int32, (1, NPIV), 1)
    cmp = (kj > ki) | ((kj == ki) & (jj < ii))
    rank_ref[0] = jnp.sum(cmp.astype(jnp.int32), axis=1, keepdims=True)


def _stage2a(piv):
    B = piv.shape[0]
    out = pl.pallas_call(
        _pivrank_body,
        grid=(B, NPIV // TB),
        in_specs=[pl.BlockSpec((1, TB, 1), lambda b, ib: (b, ib, 0)),
                  pl.BlockSpec((1, 1, NPIV), lambda b, ib: (b, 0, 0))],
        out_specs=pl.BlockSpec((1, TB, 1), lambda b, ib: (b, ib, 0)),
        out_shape=jax.ShapeDtypeStruct((B, NPIV, 1), jnp.int32),
    )(piv.reshape(B, NPIV, 1), piv.reshape(B, 1, NPIV))
    return out.reshape(B, NPIV)


def _arank_body(keyc_ref, idxc_ref, keyr_ref, idxr_ref, acnt_ref, rank_ref):
    ib = pl.program_id(1)
    ki = keyc_ref[0]
    ii = idxc_ref[0]
    islot = ib * TB + jax.lax.broadcasted_iota(jnp.int32, (TB, 1), 0)
    acv = jnp.max(acnt_ref[0])

    def chunk(c, acc):
        kj = keyr_ref[0, :, pl.ds(c * RCHUNK, RCHUNK)]
        jj = idxr_ref[0, :, pl.ds(c * RCHUNK, RCHUNK)]
        jslot = c * RCHUNK + jax.lax.broadcasted_iota(jnp.int32, (1, RCHUNK), 1)
        cmp = ((kj > ki) | ((kj == ki) & (jj < ii))) & (jslot < acv)
        return acc + jnp.sum(cmp.astype(jnp.int32), axis=1, keepdims=True)

    zero = jnp.zeros((TB, 1), jnp.int32)
    nch = (jnp.minimum(acv, C) + (RCHUNK - 1)) // RCHUNK
    acc = jax.lax.cond(ib * TB < acv,
                       lambda: jax.lax.fori_loop(0, nch, chunk, zero),
                       lambda: zero)
    rank_ref[0] = jnp.where(islot < acv, acc, K + islot)


def _stage2b(akey, aidx, acnt):
    B = akey.shape[0]
    out = pl.pallas_call(
        _arank_body,
        grid=(B, C // TB),
        in_specs=[pl.BlockSpec((1, TB, 1), lambda b, ib: (b, ib, 0)),
                  pl.BlockSpec((1, TB, 1), lambda b, ib: (b, ib, 0)),
                  pl.BlockSpec((1, 1, C), lambda b, ib: (b, 0, 0)),
                  pl.BlockSpec((1, 1, C), lambda b, ib: (b, 0, 0)),
                  pl.BlockSpec((1, 1, 16), lambda b, ib: (b, 0, 0))],
        out_specs=pl.BlockSpec((1, TB, 1), lambda b, ib: (b, ib, 0)),
        out_shape=jax.ShapeDtypeStruct((B, C, 1), jnp.int32),
    )(akey.reshape(B, C, 1), aidx.reshape(B, C, 1),
      akey.reshape(B, 1, C), aidx.reshape(B, 1, C), acnt.reshape(B, 1, 16))
    return out.reshape(B, C)


def _nms_body(icols_ref, tblT_ref, keep_ref, alive_s):
    kb = pl.program_id(1)
    i0 = kb * TBN
    lane1 = jax.lax.broadcasted_iota(jnp.int32, (1, NCHUNK), 1)
    laneT = jax.lax.broadcasted_iota(jnp.int32, (1, TBN), 1)

    @pl.when(kb == 0)
    def _init():
        alive_s[0:1, :] = (tblT_ref[0, 7:8, :] > 0.0).astype(jnp.float32)

    xi = icols_ref[0, :, 0:1]
    yi = icols_ref[0, :, 1:2]
    zi = icols_ref[0, :, 2:3]
    hxi = jnp.abs(icols_ref[0, :, 3:4]) * 0.5
    hyi = jnp.abs(icols_ref[0, :, 4:5]) * 0.5
    hzi = jnp.abs(icols_ref[0, :, 5:6]) * 0.5
    voli = hxi * hyi * hzi * 8.0

    def iou_tile(xj, yj, zj, hxj, hyj, hzj, volj):
        ix = jnp.maximum(jnp.minimum(xi + hxi, xj + hxj) - jnp.maximum(xi - hxi, xj - hxj), 0.0)
        iy = jnp.maximum(jnp.minimum(yi + hyi, yj + hyj) - jnp.maximum(yi - hyi, yj - hyj), 0.0)
        iz = jnp.maximum(jnp.minimum(zi + hzi, zj + hzj) - jnp.maximum(zi - hzi, zj - hzj), 0.0)
        inter = ix * iy * iz
        return inter / (voli + volj - inter + 1e-8)

    def planes(j0, w):
        xj = tblT_ref[0, 0:1, pl.ds(j0, w)]
        yj = tblT_ref[0, 1:2, pl.ds(j0, w)]
        zj = tblT_ref[0, 2:3, pl.ds(j0, w)]
        hxj = jnp.abs(tblT_ref[0, 3:4, pl.ds(j0, w)]) * 0.5
        hyj = jnp.abs(tblT_ref[0, 4:5, pl.ds(j0, w)]) * 0.5
        hzj = jnp.abs(tblT_ref[0, 5:6, pl.ds(j0, w)]) * 0.5
        return xj, yj, zj, hxj, hyj, hzj, hxj * hyj * hzj * 8.0

    rowT = jax.lax.broadcasted_iota(jnp.int32, (TBN, TBN), 0)
    colT = jax.lax.broadcasted_iota(jnp.int32, (TBN, TBN), 1)
    mup = jnp.where((iou_tile(*planes(i0, TBN)) > NMS_THRESH) & (colT > rowT),
                    1.0, 0.0)
    valid_loc = alive_s[0:1, pl.ds(i0, TBN)]

    def jac_cond(c):
        _, changed, it = c
        return changed & (it < TBN + 2)

    def jac_body(c):
        al, _, it = c
        sup = jnp.dot(al, mup, preferred_element_type=jnp.float32)
        new = jnp.where(sup > 0.5, 0.0, valid_loc)
        return new, jnp.sum(jnp.abs(new - al)) > 0.0, it + 1

    al, _, _ = jax.lax.while_loop(
        jac_cond, jac_body, (valid_loc, jnp.bool_(True), jnp.int32(0)))
    alive_s[0:1, pl.ds(i0, TBN)] = al

    def chunk(c, _):
        j0 = c * NCHUNK

        @pl.when(j0 + NCHUNK > i0 + TBN)
        def _():
            Mc = (iou_tile(*planes(j0, NCHUNK)) > NMS_THRESH).astype(jnp.float32)
            sup = jnp.dot(al, Mc, preferred_element_type=jnp.float32)
            later = (j0 + lane1) >= i0 + TBN
            hit = (sup > 0.5) & later
            cur = alive_s[0:1, pl.ds(j0, NCHUNK)]
            alive_s[0:1, pl.ds(j0, NCHUNK)] = jnp.where(hit, 0.0, cur)
        return 0

    jax.lax.fori_loop(0, K // NCHUNK, chunk, 0)
    keep_ref[0, 0:1, :] = alive_s[0:1, :]


def _stage4(icols, tblT):
    B = icols.shape[0]
    return pl.pallas_call(
        _nms_body,
        grid=(B, K // TBN),
        in_specs=[pl.BlockSpec((1, TBN, 8), lambda b, kb: (b, kb, 0)),
                  pl.BlockSpec((1, 8, K), lambda b, kb: (b, 0, 0))],
        out_specs=pl.BlockSpec((1, 1, K), lambda b, kb: (b, 0, 0)),
        out_shape=jax.ShapeDtypeStruct((B, 1, K), jnp.float32),
        scratch_shapes=[pltpu.VMEM((1, K), jnp.float32)],
    )(icols, tblT)


_NC, _NS, _NW = 2, 16, 32
_SPAD = K + C
_B = 4
_ATR = 2 * NPAD

_sc_mesh = plsc.VectorSubcoreMesh(
    core_axis_name="c", subcore_axis_name="s", num_cores=_NC, num_subcores=_NS)

_SC_PARAMS = pltpu.CompilerParams(use_tc_tiling_on_sc=False,
                                  needs_layout_passes=False)


_CHA = NPAD // _NS


def _sc_compact_body(key_hbm, table_hbm, tk_hbm, ti_hbm, atbl_hbm, acnt_hbm,
                     keys_v, mask_v, rows_v, idx2d, tk_v, ti_v, acv_v,
                     ctr_smem, sem):
    cid = lax.axis_index("c")
    sid = lax.axis_index("s")

    @pl.when(sid == 0)
    def _():
        ctr_smem[0] = 0
        ctr_smem[1] = 0

    plsc.subcore_barrier()

    for b2 in range(2):
        b = cid * 2 + b2
        base = b * NPAD + sid * _CHA
        pltpu.sync_copy(key_hbm.at[pl.ds(base, _CHA)], keys_v)
        pltpu.sync_copy(tk_hbm.at[b], tk_v)
        pltpu.sync_copy(ti_hbm.at[b], ti_v)
        tk = jnp.max(tk_v[...])
        ti = jnp.max(ti_v[...])

        acc = jnp.zeros((16,), jnp.int32)
        for v in range(_CHA // 16):
            k16 = keys_v[pl.ds(v * 16, 16)]
            gi = sid * _CHA + v * 16 + lax.iota(jnp.int32, 16)
            m = (k16 > tk) | ((k16 == tk) & (gi <= ti))
            mask_v[pl.ds(v * 16, 16)] = jnp.where(m, 1, 0)
            acc = acc + jnp.where(m, 1, 0)
        cnt = jnp.sum(acc)

        base_slot = plsc.fetch_and_add(ctr_smem.at[b2], cnt, subcore_id=0)

        run = base_slot
        for v in range(_CHA // 16):
            mv = mask_v[pl.ds(v * 16, 16)]
            pref = plsc.cumsum(mv)
            slotg = sid * _CHA + v * 16 + lax.iota(jnp.int32, 16)
            dst = jnp.where(mv > 0, run + pref - 1, NPAD + slotg) + b * _ATR
            idx2d[v // 8, pl.ds((v % 8) * 16, 16)] = dst
            run = run + jnp.max(pref)

        pltpu.sync_copy(table_hbm.at[pl.ds(base, _CHA), :], rows_v)
        cps = [pltpu.async_copy(rows_v.at[pl.ds(j * 128, 128), :],
                                atbl_hbm.at[idx2d.at[j]], sem)
               for j in range(_CHA // 128)]
        for cp in cps:
            cp.wait()

    plsc.subcore_barrier()

    @pl.when(sid == 0)
    def _():
        for b2 in range(2):
            b = cid * 2 + b2
            acv_v[...] = jnp.zeros((16,), jnp.int32) + ctr_smem[b2]
            pltpu.sync_copy(acv_v, acnt_hbm.at[b])


def _stageA(key_flat, table_flat, tk16, ti16):
    return pl.kernel(
        _sc_compact_body,
        out_type=[jax.ShapeDtypeStruct((_B * _ATR, 16), jnp.float32),
                  jax.ShapeDtypeStruct((_B, 16), jnp.int32)],
        mesh=_sc_mesh,
        compiler_params=_SC_PARAMS,
        scratch_types=[pltpu.VMEM((_CHA,), jnp.int32),
                       pltpu.VMEM((_CHA,), jnp.int32),
                       pltpu.VMEM((_CHA, 16), jnp.float32),
                       pltpu.VMEM((_CHA // 128, 128), jnp.int32),
                       pltpu.VMEM((16,), jnp.int32),
                       pltpu.VMEM((16,), jnp.int32),
                       pltpu.VMEM((16,), jnp.int32),
                       pltpu.SMEM((8,), jnp.int32),
                       pltpu.SemaphoreType.DMA],
    )(key_flat, table_flat, tk16, ti16)


_CH3 = C // _NS


def _sc_scatter_body(rank_hbm, atbl_hbm, sorted_hbm, rank_v, rows_v, idx2d, sem):
    cid = lax.axis_index("c")
    sid = lax.axis_index("s")
    for b2 in range(2):
        b = cid * 2 + b2
        rbase = b * C + sid * _CH3
        tbase = b * _ATR + sid * _CH3
        pltpu.sync_copy(rank_hbm.at[pl.ds(rbase, _CH3)], rank_v)
        pltpu.sync_copy(atbl_hbm.at[pl.ds(tbase, _CH3), :], rows_v)
        for j in range(_CH3 // 128):
            for i in range(8):
                off = j * 128 + i * 16
                r = rank_v[pl.ds(off, 16)]
                trash = K + sid * _CH3 + off + lax.iota(jnp.int32, 16)
                idx2d[j, pl.ds(i * 16, 16)] = jnp.where(r < K, r, trash) + b * _SPAD
        cps = [pltpu.async_copy(rows_v.at[pl.ds(j * 128, 128), :],
                                sorted_hbm.at[idx2d.at[j]], sem)
               for j in range(_CH3 // 128)]
        for cp in cps:
            cp.wait()


def _stage3(rank_flat, atbl_flat):
    return pl.kernel(
        _sc_scatter_body,
        out_type=jax.ShapeDtypeStruct((_B * _SPAD, 16), jnp.float32),
        mesh=_sc_mesh,
        compiler_params=_SC_PARAMS,
        scratch_types=[pltpu.VMEM((_CH3,), jnp.int32),
                       pltpu.VMEM((_CH3, 16), jnp.float32),
                       pltpu.VMEM((_CH3 // 128, 128), jnp.int32),
                       pltpu.SemaphoreType.DMA],
    )(rank_flat, atbl_flat)


_OUTP = 512


def _sc_select_body(keep_hbm, sorted_hbm, fb_hbm, fs_hbm, fl_hbm,
                    keep_v, sel_v, idx2d, rows_v, sc_v, lb_v, sem):
    wid = lax.axis_index("s") * _NC + lax.axis_index("c")
    b = wid

    @pl.when(wid < _B)
    def _():
        pltpu.sync_copy(keep_hbm.at[pl.ds(b * K, K)], keep_v)

        def comp(i, cnt):
            kv = keep_v[pl.ds(i * 16, 16)]
            mask = kv > 0.5
            pref = plsc.cumsum(jnp.where(mask, 1, 0))
            posv = i * 16 + lax.iota(jnp.int32, 16)
            dst = jnp.where(mask, cnt + pref - 1, K + 8)
            plsc.store_scatter(sel_v, [dst], posv)
            return cnt + jnp.max(pref)

        count = lax.fori_loop(0, K // 16, comp, 0)

        for i in range(_OUTP // 16):
            sl = sel_v[pl.ds(i * 16, 16)]
            slot = i * 16 + lax.iota(jnp.int32, 16)
            idx = jnp.where(slot < count, sl, 0) + b * _SPAD
            idx2d[i // 8, pl.ds((i % 8) * 16, 16)] = idx

        cps = [pltpu.async_copy(sorted_hbm.at[idx2d.at[j]],
                                rows_v.at[pl.ds(j * 128, 128), :], sem)
               for j in range(_OUTP // 128)]
        for cp in cps:
            cp.wait()

        def zero(i, _):
            vf = jnp.where(i < count, 1.0, 0.0)
            rows_v[i] = rows_v[i] * vf
            return 0

        lax.fori_loop(0, _OUTP, zero, 0)

        def cols(i, _):
            slot = i * 16 + lax.iota(jnp.int32, 16)
            sc = plsc.load_gather(rows_v, [slot, jnp.full((16,), 7, jnp.int32)])
            lb = plsc.load_gather(rows_v, [slot, jnp.full((16,), 8, jnp.int32)])
            sc_v[pl.ds(i * 16, 16)] = sc
            lb_v[pl.ds(i * 16, 16)] = lb.astype(jnp.int32)
            return 0

        lax.fori_loop(0, _OUTP // 16, cols, 0)

        pltpu.sync_copy(rows_v, fb_hbm.at[pl.ds(b * _OUTP, _OUTP), :])
        pltpu.sync_copy(sc_v, fs_hbm.at[pl.ds(b * _OUTP, _OUTP)])
        pltpu.sync_copy(lb_v, fl_hbm.at[pl.ds(b * _OUTP, _OUTP)])


def _stage5(keep_flat, sorted_flat):
    return pl.kernel(
        _sc_select_body,
        out_type=[jax.ShapeDtypeStruct((_B * _OUTP, 16), jnp.float32),
                  jax.ShapeDtypeStruct((_B * _OUTP,), jnp.float32),
                  jax.ShapeDtypeStruct((_B * _OUTP,), jnp.int32)],
        mesh=_sc_mesh,
        compiler_params=pltpu.CompilerParams(use_tc_tiling_on_sc=False, needs_layout_passes=False),
        scratch_types=[pltpu.VMEM((K,), jnp.float32),
                       pltpu.VMEM((K + 16,), jnp.int32),
                       pltpu.VMEM((_OUTP // 128, 128), jnp.int32),
                       pltpu.VMEM((_OUTP, 16), jnp.float32),
                       pltpu.VMEM((_OUTP,), jnp.float32),
                       pltpu.VMEM((_OUTP,), jnp.int32),
                       pltpu.SemaphoreType.DMA],
    )(keep_flat, sorted_flat)


@jax.jit
def _run(batch_box_preds, batch_cls_preds):
    B, n, _ = batch_box_preds.shape
    cls3d = jnp.pad(batch_cls_preds, ((0, 0), (0, NPAD - n), (0, 0)),
                    constant_values=-30.0).transpose(0, 2, 1)
    boxes = jnp.pad(batch_box_preds, ((0, 0), (0, NPAD - n), (0, 0)))

    masked3, label3 = _stage1(cls3d)
    masked = masked3.reshape(B, NPAD)
    label = label3.reshape(B, NPAD)
    key = jax.lax.bitcast_convert_type(masked, jnp.int32)

    piv = key[:, ::PSTRIDE]
    rankp = _stage2a(piv)
    sel = rankp == S_SEL
    tkey = jnp.sum(jnp.where(sel, piv, 0), axis=1)
    tm = jnp.sum(jnp.where(sel, jnp.arange(NPIV, dtype=jnp.int32)[None, :], 0),
                 axis=1)
    tk16 = jnp.broadcast_to(tkey[:, None], (B, 16)).astype(jnp.int32)
    ti16 = jnp.broadcast_to((tm * PSTRIDE)[:, None], (B, 16)).astype(jnp.int32)

    oidx = jnp.broadcast_to(
        jnp.arange(NPAD, dtype=jnp.float32)[None, :, None], (B, NPAD, 1))
    table = jnp.concatenate(
        [boxes, masked[..., None], label.astype(jnp.float32)[..., None],
         oidx, jnp.zeros((B, NPAD, 6), jnp.float32)], axis=-1)

    atbl_flat, acnt = _stageA(key.reshape(B * NPAD),
                              table.reshape(B * NPAD, 16), tk16, ti16)
    atbl4 = atbl_flat.reshape(B, _ATR, 16)
    akey = jax.lax.bitcast_convert_type(atbl4[:, :C, 7], jnp.int32)
    aidx = atbl4[:, :C, 9].astype(jnp.int32)
    rankA = _stage2b(akey, aidx, acnt)

    sorted_flat = _stage3(rankA.reshape(B * C), atbl_flat)
    sorted4 = sorted_flat.reshape(B, _SPAD, 16)
    icols = sorted4[:, :K, :8]
    tblT = icols.transpose(0, 2, 1)

    keep3 = _stage4(icols, tblT)

    fbf, fsf, flf = _stage5(keep3.reshape(B * K), sorted_flat)
    fb = fbf.reshape(B, _OUTP, 16)[:, :OUT, :7]
    fs = fsf.reshape(B, _OUTP)[:, :OUT]
    fl = flf.reshape(B, _OUTP)[:, :OUT]
    return fb, fs, fl


def kernel(batch_box_preds, batch_cls_preds):
    return _run(batch_box_preds, batch_cls_preds)


# speedup vs baseline: 172.8580x; 1.0515x over previous
import jax
import jax.numpy as jnp
from jax import lax
from jax.experimental import pallas as pl
from jax.experimental.pallas import tpu as pltpu
from jax.experimental.pallas import tpu_sc as plsc

SCORE_THRESH = 0.1
NMS_THRESH = 0.01
K = 4096
OUT = 500
NPAD = 20480
TB = 512
TBN = 512
RCHUNK = 2048
NCHUNK = 512

PSTRIDE = 20
NPIV = NPAD // PSTRIDE
S_SEL = 304
C = 8192


def _scores_body(cls_ref, masked_ref, label_ref):
    c0 = cls_ref[0, 0:1, :]
    c1 = cls_ref[0, 1:2, :]
    c2 = cls_ref[0, 2:3, :]
    s0 = jax.nn.sigmoid(c0)
    s1 = jax.nn.sigmoid(c1)
    s2 = jax.nn.sigmoid(c2)
    m01 = jnp.maximum(s0, s1)
    i01 = jnp.where(s1 > s0, 1, 0)
    score = jnp.maximum(m01, s2)
    label = jnp.where(s2 > m01, 2, i01) + 1
    masked = jnp.where(score >= SCORE_THRESH, score, -1.0)
    masked_ref[0, :, :] = masked
    label_ref[0, :, :] = label


def _stage1(cls3d):
    B = cls3d.shape[0]
    return pl.pallas_call(
        _scores_body,
        grid=(B,),
        in_specs=[pl.BlockSpec((1, 3, NPAD), lambda b: (b, 0, 0))],
        out_specs=[pl.BlockSpec((1, 1, NPAD), lambda b: (b, 0, 0)),
                   pl.BlockSpec((1, 1, NPAD), lambda b: (b, 0, 0))],
        out_shape=[jax.ShapeDtypeStruct((B, 1, NPAD), jnp.float32),
                   jax.ShapeDtypeStruct((B, 1, NPAD), jnp.int32)],
    )(cls3d)


def _pivrank_body(keyc_ref, keyr_ref, rank_ref):
    ib = pl.program_id(1)
    ki = keyc_ref[0]
    ii = ib * TB + jax.lax.broadcasted_iota(jnp.int32, (TB, 1), 0)
    kj = keyr_ref[0]
    jj = jax.lax.broadcasted_iota(jnp.int32, (1, NPIV), 1)
    cmp = (kj > ki) | ((kj == ki) & (jj < ii))
    rank_ref[0] = jnp.sum(cmp.astype(jnp.int32), axis=1, keepdims=True)


def _stage2a(piv):
    B = piv.shape[0]
    out = pl.pallas_call(
        _pivrank_body,
        grid=(B, NPIV // TB),
        in_specs=[pl.BlockSpec((1, TB, 1), lambda b, ib: (b, ib, 0)),
                  pl.BlockSpec((1, 1, NPIV), lambda b, ib: (b, 0, 0))],
        out_specs=pl.BlockSpec((1, TB, 1), lambda b, ib: (b, ib, 0)),
        out_shape=jax.ShapeDtypeStruct((B, NPIV, 1), jnp.int32),
    )(piv.reshape(B, NPIV, 1), piv.reshape(B, 1, NPIV))
    return out.reshape(B, NPIV)


def _arank_body(keyc_ref, idxc_ref, keyr_ref, idxr_ref, acnt_ref, rank_ref):
    ib = pl.program_id(1)
    ki = keyc_ref[0]
    ii = idxc_ref[0]
    islot = ib * TB + jax.lax.broadcasted_iota(jnp.int32, (TB, 1), 0)
    acv = jnp.max(acnt_ref[0])

    def chunk(c, acc):
        kj = keyr_ref[0, :, pl.ds(c * RCHUNK, RCHUNK)]
        jj = idxr_ref[0, :, pl.ds(c * RCHUNK, RCHUNK)]
        jslot = c * RCHUNK + jax.lax.broadcasted_iota(jnp.int32, (1, RCHUNK), 1)
        cmp = ((kj > ki) | ((kj == ki) & (jj < ii))) & (jslot < acv)
        return acc + jnp.sum(cmp.astype(jnp.int32), axis=1, keepdims=True)

    zero = jnp.zeros((TB, 1), jnp.int32)
    nch = (jnp.minimum(acv, C) + (RCHUNK - 1)) // RCHUNK
    acc = jax.lax.cond(ib * TB < acv,
                       lambda: jax.lax.fori_loop(0, nch, chunk, zero),
                       lambda: zero)
    rank_ref[0] = jnp.where(islot < acv, acc, K + islot)


def _stage2b(akey, aidx, acnt):
    B = akey.shape[0]
    out = pl.pallas_call(
        _arank_body,
        grid=(B, C // TB),
        in_specs=[pl.BlockSpec((1, TB, 1), lambda b, ib: (b, ib, 0)),
                  pl.BlockSpec((1, TB, 1), lambda b, ib: (b, ib, 0)),
                  pl.BlockSpec((1, 1, C), lambda b, ib: (b, 0, 0)),
                  pl.BlockSpec((1, 1, C), lambda b, ib: (b, 0, 0)),
                  pl.BlockSpec((1, 1, 16), lambda b, ib: (b, 0, 0))],
        out_specs=pl.BlockSpec((1, TB, 1), lambda b, ib: (b, ib, 0)),
        out_shape=jax.ShapeDtypeStruct((B, C, 1), jnp.int32),
    )(akey.reshape(B, C, 1), aidx.reshape(B, C, 1),
      akey.reshape(B, 1, C), aidx.reshape(B, 1, C), acnt.reshape(B, 1, 16))
    return out.reshape(B, C)


def _nms_body(icols_ref, tblT_ref, keep_ref, alive_s):
    kb = pl.program_id(1)
    i0 = kb * TBN
    lane1 = jax.lax.broadcasted_iota(jnp.int32, (1, NCHUNK), 1)
    laneT = jax.lax.broadcasted_iota(jnp.int32, (1, TBN), 1)

    @pl.when(kb == 0)
    def _init():
        alive_s[0:1, :] = (tblT_ref[0, 7:8, :] > 0.0).astype(jnp.float32)

    xi = icols_ref[0, :, 0:1]
    yi = icols_ref[0, :, 1:2]
    zi = icols_ref[0, :, 2:3]
    hxi = jnp.abs(icols_ref[0, :, 3:4]) * 0.5
    hyi = jnp.abs(icols_ref[0, :, 4:5]) * 0.5
    hzi = jnp.abs(icols_ref[0, :, 5:6]) * 0.5
    voli = hxi * hyi * hzi * 8.0

    def iou_tile(xj, yj, zj, hxj, hyj, hzj, volj):
        ix = jnp.maximum(jnp.minimum(xi + hxi, xj + hxj) - jnp.maximum(xi - hxi, xj - hxj), 0.0)
        iy = jnp.maximum(jnp.minimum(yi + hyi, yj + hyj) - jnp.maximum(yi - hyi, yj - hyj), 0.0)
        iz = jnp.maximum(jnp.minimum(zi + hzi, zj + hzj) - jnp.maximum(zi - hzi, zj - hzj), 0.0)
        inter = ix * iy * iz
        return inter / (voli + volj - inter + 1e-8)

    def planes(j0, w):
        xj = tblT_ref[0, 0:1, pl.ds(j0, w)]
        yj = tblT_ref[0, 1:2, pl.ds(j0, w)]
        zj = tblT_ref[0, 2:3, pl.ds(j0, w)]
        hxj = jnp.abs(tblT_ref[0, 3:4, pl.ds(j0, w)]) * 0.5
        hyj = jnp.abs(tblT_ref[0, 4:5, pl.ds(j0, w)]) * 0.5
        hzj = jnp.abs(tblT_ref[0, 5:6, pl.ds(j0, w)]) * 0.5
        return xj, yj, zj, hxj, hyj, hzj, hxj * hyj * hzj * 8.0

    rowT = jax.lax.broadcasted_iota(jnp.int32, (TBN, TBN), 0)
    colT = jax.lax.broadcasted_iota(jnp.int32, (TBN, TBN), 1)
    mup = jnp.where((iou_tile(*planes(i0, TBN)) > NMS_THRESH) & (colT > rowT),
                    1.0, 0.0)
    valid_loc = alive_s[0:1, pl.ds(i0, TBN)]

    def jac_cond(c):
        _, changed, it = c
        return changed & (it < TBN + 2)

    def jac_body(c):
        al, _, it = c
        sup = jnp.dot(al, mup, preferred_element_type=jnp.float32)
        new = jnp.where(sup > 0.5, 0.0, valid_loc)
        return new, jnp.sum(jnp.abs(new - al)) > 0.0, it + 1

    al, _, _ = jax.lax.while_loop(
        jac_cond, jac_body, (valid_loc, jnp.bool_(True), jnp.int32(0)))
    alive_s[0:1, pl.ds(i0, TBN)] = al

    def chunk(c, _):
        j0 = c * NCHUNK

        @pl.when(j0 + NCHUNK > i0 + TBN)
        def _():
            Mc = (iou_tile(*planes(j0, NCHUNK)) > NMS_THRESH).astype(jnp.float32)
            sup = jnp.dot(al, Mc, preferred_element_type=jnp.float32)
            later = (j0 + lane1) >= i0 + TBN
            hit = (sup > 0.5) & later
            cur = alive_s[0:1, pl.ds(j0, NCHUNK)]
            alive_s[0:1, pl.ds(j0, NCHUNK)] = jnp.where(hit, 0.0, cur)
        return 0

    jax.lax.fori_loop(0, K // NCHUNK, chunk, 0)
    keep_ref[0, 0:1, :] = alive_s[0:1, :]


def _stage4(icols, tblT):
    B = icols.shape[0]
    return pl.pallas_call(
        _nms_body,
        grid=(B, K // TBN),
        in_specs=[pl.BlockSpec((1, TBN, 8), lambda b, kb: (b, kb, 0)),
                  pl.BlockSpec((1, 8, K), lambda b, kb: (b, 0, 0))],
        out_specs=pl.BlockSpec((1, 1, K), lambda b, kb: (b, 0, 0)),
        out_shape=jax.ShapeDtypeStruct((B, 1, K), jnp.float32),
        scratch_shapes=[pltpu.VMEM((1, K), jnp.float32)],
    )(icols, tblT)


_NC, _NS, _NW = 2, 16, 32
_SPAD = K + C
_B = 4
_ATR = 2 * NPAD

_sc_mesh = plsc.VectorSubcoreMesh(
    core_axis_name="c", subcore_axis_name="s", num_cores=_NC, num_subcores=_NS)

_SC_PARAMS = pltpu.CompilerParams(use_tc_tiling_on_sc=False,
                                  needs_layout_passes=False)


_CHA = NPAD // _NS


def _sc_compact_body(key_hbm, table_hbm, tk_hbm, ti_hbm, atbl_hbm, acnt_hbm,
                     keys_v, mask_v, rows_v, idx2d, tk_v, ti_v, acv_v,
                     ctr_smem, sem):
    cid = lax.axis_index("c")
    sid = lax.axis_index("s")

    @pl.when(sid == 0)
    def _():
        ctr_smem[0] = 0
        ctr_smem[1] = 0

    plsc.subcore_barrier()

    for b2 in range(2):
        b = cid * 2 + b2
        base = b * NPAD + sid * _CHA
        pltpu.sync_copy(key_hbm.at[pl.ds(base, _CHA)], keys_v)
        pltpu.sync_copy(tk_hbm.at[b], tk_v)
        pltpu.sync_copy(ti_hbm.at[b], ti_v)
        tk = jnp.max(tk_v[...])
        ti = jnp.max(ti_v[...])

        acc = jnp.zeros((16,), jnp.int32)
        for v in range(_CHA // 16):
            k16 = keys_v[pl.ds(v * 16, 16)]
            gi = sid * _CHA + v * 16 + lax.iota(jnp.int32, 16)
            m = (k16 > tk) | ((k16 == tk) & (gi <= ti))
            mask_v[pl.ds(v * 16, 16)] = jnp.where(m, 1, 0)
            acc = acc + jnp.where(m, 1, 0)
        cnt = jnp.sum(acc)

        base_slot = plsc.fetch_and_add(ctr_smem.at[b2], cnt, subcore_id=0)

        run = base_slot
        for v in range(_CHA // 16):
            mv = mask_v[pl.ds(v * 16, 16)]
            pref = plsc.cumsum(mv)
            slotg = sid * _CHA + v * 16 + lax.iota(jnp.int32, 16)
            dst = jnp.where(mv > 0, run + pref - 1, NPAD + slotg) + b * _ATR
            idx2d[v // 8, pl.ds((v % 8) * 16, 16)] = dst
            run = run + jnp.max(pref)

        pltpu.sync_copy(table_hbm.at[pl.ds(base, _CHA), :], rows_v)
        cps = [pltpu.async_copy(rows_v.at[pl.ds(j * 128, 128), :],
                                atbl_hbm.at[idx2d.at[j]], sem)
               for j in range(_CHA // 128)]
        for cp in cps:
            cp.wait()

    plsc.subcore_barrier()

    @pl.when(sid == 0)
    def _():
        for b2 in range(2):
            b = cid * 2 + b2
            acv_v[...] = jnp.zeros((16,), jnp.int32) + ctr_smem[b2]
            pltpu.sync_copy(acv_v, acnt_hbm.at[b])


def _stageA(key_flat, table_flat, tk16, ti16):
    return pl.kernel(
        _sc_compact_body,
        out_type=[jax.ShapeDtypeStruct((_B * _ATR, 16), jnp.float32),
                  jax.ShapeDtypeStruct((_B, 16), jnp.int32)],
        mesh=_sc_mesh,
        compiler_params=_SC_PARAMS,
        scratch_types=[pltpu.VMEM((_CHA,), jnp.int32),
                       pltpu.VMEM((_CHA,), jnp.int32),
                       pltpu.VMEM((_CHA, 16), jnp.float32),
                       pltpu.VMEM((_CHA // 128, 128), jnp.int32),
                       pltpu.VMEM((16,), jnp.int32),
                       pltpu.VMEM((16,), jnp.int32),
                       pltpu.VMEM((16,), jnp.int32),
                       pltpu.SMEM((8,), jnp.int32),
                       pltpu.SemaphoreType.DMA],
    )(key_flat, table_flat, tk16, ti16)


_CH3 = C // _NS


def _sc_scatter_body(rank_hbm, atbl_hbm, sorted_hbm, rank_v, rows_v, idx2d, sem):
    cid = lax.axis_index("c")
    sid = lax.axis_index("s")
    for b2 in range(2):
        b = cid * 2 + b2
        rbase = b * C + sid * _CH3
        tbase = b * _ATR + sid * _CH3
        pltpu.sync_copy(rank_hbm.at[pl.ds(rbase, _CH3)], rank_v)
        pltpu.sync_copy(atbl_hbm.at[pl.ds(tbase, _CH3), :], rows_v)
        for j in range(_CH3 // 128):
            for i in range(8):
                off = j * 128 + i * 16
                r = rank_v[pl.ds(off, 16)]
                trash = K + sid * _CH3 + off + lax.iota(jnp.int32, 16)
                idx2d[j, pl.ds(i * 16, 16)] = jnp.where(r < K, r, trash) + b * _SPAD
        cps = [pltpu.async_copy(rows_v.at[pl.ds(j * 128, 128), :],
                                sorted_hbm.at[idx2d.at[j]], sem)
               for j in range(_CH3 // 128)]
        for cp in cps:
            cp.wait()


def _stage3(rank_flat, atbl_flat):
    return pl.kernel(
        _sc_scatter_body,
        out_type=jax.ShapeDtypeStruct((_B * _SPAD, 16), jnp.float32),
        mesh=_sc_mesh,
        compiler_params=_SC_PARAMS,
        scratch_types=[pltpu.VMEM((_CH3,), jnp.int32),
                       pltpu.VMEM((_CH3, 16), jnp.float32),
                       pltpu.VMEM((_CH3 // 128, 128), jnp.int32),
                       pltpu.SemaphoreType.DMA],
    )(rank_flat, atbl_flat)


_OUTP = 512


def _sc_select_body(keep_hbm, sorted_hbm, fb_hbm, fs_hbm, fl_hbm,
                    keep_v, sel_v, idx2d, rows_v, sc_v, lb_v, sem):
    wid = lax.axis_index("s") * _NC + lax.axis_index("c")
    b = wid

    @pl.when(wid < _B)
    def _():
        pltpu.sync_copy(keep_hbm.at[pl.ds(b * K, K)], keep_v)

        def comp(i, cnt):
            kv = keep_v[pl.ds(i * 16, 16)]
            mask = kv > 0.5
            pref = plsc.cumsum(jnp.where(mask, 1, 0))
            posv = i * 16 + lax.iota(jnp.int32, 16)
            dst = jnp.where(mask, cnt + pref - 1, K + 8)
            plsc.store_scatter(sel_v, [dst], posv)
            return cnt + jnp.max(pref)

        count = lax.fori_loop(0, K // 16, comp, 0)

        for i in range(_OUTP // 16):
            sl = sel_v[pl.ds(i * 16, 16)]
            slot = i * 16 + lax.iota(jnp.int32, 16)
            idx = jnp.where(slot < count, sl, 0) + b * _SPAD
            idx2d[i // 8, pl.ds((i % 8) * 16, 16)] = idx

        cps = [pltpu.async_copy(sorted_hbm.at[idx2d.at[j]],
                                rows_v.at[pl.ds(j * 128, 128), :], sem)
               for j in range(_OUTP // 128)]
        for cp in cps:
            cp.wait()

        def zero(i, _):
            vf = jnp.where(i < count, 1.0, 0.0)
            rows_v[i] = rows_v[i] * vf
            return 0

        lax.fori_loop(0, _OUTP, zero, 0)

        def cols(i, _):
            slot = i * 16 + lax.iota(jnp.int32, 16)
            sc = plsc.load_gather(rows_v, [slot, jnp.full((16,), 7, jnp.int32)])
            lb = plsc.load_gather(rows_v, [slot, jnp.full((16,), 8, jnp.int32)])
            sc_v[pl.ds(i * 16, 16)] = sc
            lb_v[pl.ds(i * 16, 16)] = lb.astype(jnp.int32)
            return 0

        lax.fori_loop(0, _OUTP // 16, cols, 0)

        pltpu.sync_copy(rows_v, fb_hbm.at[pl.ds(b * _OUTP, _OUTP), :])
        pltpu.sync_copy(sc_v, fs_hbm.at[pl.ds(b * _OUTP, _OUTP)])
        pltpu.sync_copy(lb_v, fl_hbm.at[pl.ds(b * _OUTP, _OUTP)])


def _stage5(keep_flat, sorted_flat):
    return pl.kernel(
        _sc_select_body,
        out_type=[jax.ShapeDtypeStruct((_B * _OUTP, 16), jnp.float32),
                  jax.ShapeDtypeStruct((_B * _OUTP,), jnp.float32),
                  jax.ShapeDtypeStruct((_B * _OUTP,), jnp.int32)],
        mesh=_sc_mesh,
        compiler_params=pltpu.CompilerParams(use_tc_tiling_on_sc=False, needs_layout_passes=False),
        scratch_types=[pltpu.VMEM((K,), jnp.float32),
                       pltpu.VMEM((K + 16,), jnp.int32),
                       pltpu.VMEM((_OUTP // 128, 128), jnp.int32),
                       pltpu.VMEM((_OUTP, 16), jnp.float32),
                       pltpu.VMEM((_OUTP,), jnp.float32),
                       pltpu.VMEM((_OUTP,), jnp.int32),
                       pltpu.SemaphoreType.DMA],
    )(keep_flat, sorted_flat)


@jax.jit
def _run(batch_box_preds, batch_cls_preds):
    B, n, _ = batch_box_preds.shape
    cls3d = jnp.pad(batch_cls_preds, ((0, 0), (0, NPAD - n), (0, 0)),
                    constant_values=-30.0).transpose(0, 2, 1)
    boxes = jnp.pad(batch_box_preds, ((0, 0), (0, NPAD - n), (0, 0)))

    masked3, label3 = _stage1(cls3d)
    masked = masked3.reshape(B, NPAD)
    label = label3.reshape(B, NPAD)
    key = jax.lax.bitcast_convert_type(masked, jnp.int32)

    piv = key[:, ::PSTRIDE]
    rankp = _stage2a(piv)
    sel = rankp == S_SEL
    tkey = jnp.sum(jnp.where(sel, piv, 0), axis=1)
    tm = jnp.sum(jnp.where(sel, jnp.arange(NPIV, dtype=jnp.int32)[None, :], 0),
                 axis=1)
    tk16 = jnp.broadcast_to(tkey[:, None], (B, 16)).astype(jnp.int32)
    ti16 = jnp.broadcast_to((tm * PSTRIDE)[:, None], (B, 16)).astype(jnp.int32)

    oidx = jnp.broadcast_to(
        jnp.arange(NPAD, dtype=jnp.float32)[None, :, None], (B, NPAD, 1))
    table = jnp.concatenate(
        [boxes, masked[..., None], label.astype(jnp.float32)[..., None],
         oidx, jnp.zeros((B, NPAD, 6), jnp.float32)], axis=-1)

    atbl_flat, acnt = _stageA(key.reshape(B * NPAD),
                              table.reshape(B * NPAD, 16), tk16, ti16)
    atbl4 = atbl_flat.reshape(B, _ATR, 16)
    akey = jax.lax.bitcast_convert_type(atbl4[:, :C, 7], jnp.int32)
    aidx = atbl4[:, :C, 9].astype(jnp.int32)
    rankA = _stage2b(akey, aidx, acnt)

    sorted_flat = _stage3(rankA.reshape(B * C), atbl_flat)
    sorted4 = sorted_flat.reshape(B, _SPAD, 16)
    icols = sorted4[:, :K, :8]
    tblT = icols.transpose(0, 2, 1)

    keep3 = _stage4(icols, tblT)

    fbf, fsf, flf = _stage5(keep3.reshape(B * K), sorted_flat)
    fb = fbf.reshape(B, _OUTP, 16)[:, :OUT, :7]
    fs = fsf.reshape(B, _OUTP)[:, :OUT]
    fl = flf.reshape(B, _OUTP)[:, :OUT]
    return fb, fs, fl


def kernel(batch_box_preds, batch_cls_preds):
    return _run(batch_box_preds, batch_cls_preds)


# speedup vs baseline: 176.8382x; 1.0230x over previous
import jax
import jax.numpy as jnp
from jax import lax
from jax.experimental import pallas as pl
from jax.experimental.pallas import tpu as pltpu
from jax.experimental.pallas import tpu_sc as plsc

SCORE_THRESH = 0.1
NMS_THRESH = 0.01
K = 4096
OUT = 500
NPAD = 20480
TB = 512
TBN = 1024
RCHUNK = 2048
NCHUNK = 1024

PSTRIDE = 20
NPIV = NPAD // PSTRIDE
S_SEL = 304
C = 8192


def _scores_body(cls_ref, masked_ref, label_ref):
    c0 = cls_ref[0, 0:1, :]
    c1 = cls_ref[0, 1:2, :]
    c2 = cls_ref[0, 2:3, :]
    s0 = jax.nn.sigmoid(c0)
    s1 = jax.nn.sigmoid(c1)
    s2 = jax.nn.sigmoid(c2)
    m01 = jnp.maximum(s0, s1)
    i01 = jnp.where(s1 > s0, 1, 0)
    score = jnp.maximum(m01, s2)
    label = jnp.where(s2 > m01, 2, i01) + 1
    masked = jnp.where(score >= SCORE_THRESH, score, -1.0)
    masked_ref[0, :, :] = masked
    label_ref[0, :, :] = label


def _stage1(cls3d):
    B = cls3d.shape[0]
    return pl.pallas_call(
        _scores_body,
        grid=(B,),
        in_specs=[pl.BlockSpec((1, 3, NPAD), lambda b: (b, 0, 0))],
        out_specs=[pl.BlockSpec((1, 1, NPAD), lambda b: (b, 0, 0)),
                   pl.BlockSpec((1, 1, NPAD), lambda b: (b, 0, 0))],
        out_shape=[jax.ShapeDtypeStruct((B, 1, NPAD), jnp.float32),
                   jax.ShapeDtypeStruct((B, 1, NPAD), jnp.int32)],
    )(cls3d)


def _pivrank_body(keyc_ref, keyr_ref, rank_ref):
    ib = pl.program_id(1)
    ki = keyc_ref[0]
    ii = ib * TB + jax.lax.broadcasted_iota(jnp.int32, (TB, 1), 0)
    kj = keyr_ref[0]
    jj = jax.lax.broadcasted_iota(jnp.int32, (1, NPIV), 1)
    cmp = (kj > ki) | ((kj == ki) & (jj < ii))
    rank_ref[0] = jnp.sum(cmp.astype(jnp.int32), axis=1, keepdims=True)


def _stage2a(piv):
    B = piv.shape[0]
    out = pl.pallas_call(
        _pivrank_body,
        grid=(B, NPIV // TB),
        in_specs=[pl.BlockSpec((1, TB, 1), lambda b, ib: (b, ib, 0)),
                  pl.BlockSpec((1, 1, NPIV), lambda b, ib: (b, 0, 0))],
        out_specs=pl.BlockSpec((1, TB, 1), lambda b, ib: (b, ib, 0)),
        out_shape=jax.ShapeDtypeStruct((B, NPIV, 1), jnp.int32),
    )(piv.reshape(B, NPIV, 1), piv.reshape(B, 1, NPIV))
    return out.reshape(B, NPIV)


def _arank_body(keyc_ref, idxc_ref, keyr_ref, idxr_ref, acnt_ref, rank_ref):
    ib = pl.program_id(1)
    ki = keyc_ref[0]
    ii = idxc_ref[0]
    islot = ib * TB + jax.lax.broadcasted_iota(jnp.int32, (TB, 1), 0)
    acv = jnp.max(acnt_ref[0])

    def chunk(c, acc):
        kj = keyr_ref[0, :, pl.ds(c * RCHUNK, RCHUNK)]
        jj = idxr_ref[0, :, pl.ds(c * RCHUNK, RCHUNK)]
        jslot = c * RCHUNK + jax.lax.broadcasted_iota(jnp.int32, (1, RCHUNK), 1)
        cmp = ((kj > ki) | ((kj == ki) & (jj < ii))) & (jslot < acv)
        return acc + jnp.sum(cmp.astype(jnp.int32), axis=1, keepdims=True)

    zero = jnp.zeros((TB, 1), jnp.int32)
    nch = (jnp.minimum(acv, C) + (RCHUNK - 1)) // RCHUNK
    acc = jax.lax.cond(ib * TB < acv,
                       lambda: jax.lax.fori_loop(0, nch, chunk, zero),
                       lambda: zero)
    rank_ref[0] = jnp.where(islot < acv, acc, K + islot)


def _stage2b(akey, aidx, acnt):
    B = akey.shape[0]
    out = pl.pallas_call(
        _arank_body,
        grid=(B, C // TB),
        in_specs=[pl.BlockSpec((1, TB, 1), lambda b, ib: (b, ib, 0)),
                  pl.BlockSpec((1, TB, 1), lambda b, ib: (b, ib, 0)),
                  pl.BlockSpec((1, 1, C), lambda b, ib: (b, 0, 0)),
                  pl.BlockSpec((1, 1, C), lambda b, ib: (b, 0, 0)),
                  pl.BlockSpec((1, 1, 16), lambda b, ib: (b, 0, 0))],
        out_specs=pl.BlockSpec((1, TB, 1), lambda b, ib: (b, ib, 0)),
        out_shape=jax.ShapeDtypeStruct((B, C, 1), jnp.int32),
    )(akey.reshape(B, C, 1), aidx.reshape(B, C, 1),
      akey.reshape(B, 1, C), aidx.reshape(B, 1, C), acnt.reshape(B, 1, 16))
    return out.reshape(B, C)


def _nms_body(icols_ref, tblT_ref, keep_ref, alive_s):
    kb = pl.program_id(1)
    i0 = kb * TBN
    lane1 = jax.lax.broadcasted_iota(jnp.int32, (1, NCHUNK), 1)
    laneT = jax.lax.broadcasted_iota(jnp.int32, (1, TBN), 1)

    @pl.when(kb == 0)
    def _init():
        alive_s[0:1, :] = (tblT_ref[0, 7:8, :] > 0.0).astype(jnp.float32)

    xi = icols_ref[0, :, 0:1]
    yi = icols_ref[0, :, 1:2]
    zi = icols_ref[0, :, 2:3]
    hxi = jnp.abs(icols_ref[0, :, 3:4]) * 0.5
    hyi = jnp.abs(icols_ref[0, :, 4:5]) * 0.5
    hzi = jnp.abs(icols_ref[0, :, 5:6]) * 0.5
    voli = hxi * hyi * hzi * 8.0

    def iou_tile(xj, yj, zj, hxj, hyj, hzj, volj):
        ix = jnp.maximum(jnp.minimum(xi + hxi, xj + hxj) - jnp.maximum(xi - hxi, xj - hxj), 0.0)
        iy = jnp.maximum(jnp.minimum(yi + hyi, yj + hyj) - jnp.maximum(yi - hyi, yj - hyj), 0.0)
        iz = jnp.maximum(jnp.minimum(zi + hzi, zj + hzj) - jnp.maximum(zi - hzi, zj - hzj), 0.0)
        inter = ix * iy * iz
        return inter / (voli + volj - inter + 1e-8)

    def planes(j0, w):
        xj = tblT_ref[0, 0:1, pl.ds(j0, w)]
        yj = tblT_ref[0, 1:2, pl.ds(j0, w)]
        zj = tblT_ref[0, 2:3, pl.ds(j0, w)]
        hxj = jnp.abs(tblT_ref[0, 3:4, pl.ds(j0, w)]) * 0.5
        hyj = jnp.abs(tblT_ref[0, 4:5, pl.ds(j0, w)]) * 0.5
        hzj = jnp.abs(tblT_ref[0, 5:6, pl.ds(j0, w)]) * 0.5
        return xj, yj, zj, hxj, hyj, hzj, hxj * hyj * hzj * 8.0

    rowT = jax.lax.broadcasted_iota(jnp.int32, (TBN, TBN), 0)
    colT = jax.lax.broadcasted_iota(jnp.int32, (TBN, TBN), 1)
    mup = jnp.where((iou_tile(*planes(i0, TBN)) > NMS_THRESH) & (colT > rowT),
                    1.0, 0.0)
    valid_loc = alive_s[0:1, pl.ds(i0, TBN)]

    def jac_cond(c):
        _, changed, it = c
        return changed & (it < TBN + 2)

    def jac_body(c):
        al, _, it = c
        sup = jnp.dot(al, mup, preferred_element_type=jnp.float32)
        new = jnp.where(sup > 0.5, 0.0, valid_loc)
        return new, jnp.sum(jnp.abs(new - al)) > 0.0, it + 1

    al, _, _ = jax.lax.while_loop(
        jac_cond, jac_body, (valid_loc, jnp.bool_(True), jnp.int32(0)))
    alive_s[0:1, pl.ds(i0, TBN)] = al

    def chunk(c, _):
        j0 = c * NCHUNK

        @pl.when(j0 + NCHUNK > i0 + TBN)
        def _():
            Mc = (iou_tile(*planes(j0, NCHUNK)) > NMS_THRESH).astype(jnp.float32)
            sup = jnp.dot(al, Mc, preferred_element_type=jnp.float32)
            later = (j0 + lane1) >= i0 + TBN
            hit = (sup > 0.5) & later
            cur = alive_s[0:1, pl.ds(j0, NCHUNK)]
            alive_s[0:1, pl.ds(j0, NCHUNK)] = jnp.where(hit, 0.0, cur)
        return 0

    jax.lax.fori_loop(0, K // NCHUNK, chunk, 0)
    keep_ref[0, 0:1, :] = alive_s[0:1, :]


def _stage4(icols, tblT):
    B = icols.shape[0]
    return pl.pallas_call(
        _nms_body,
        grid=(B, K // TBN),
        in_specs=[pl.BlockSpec((1, TBN, 8), lambda b, kb: (b, kb, 0)),
                  pl.BlockSpec((1, 8, K), lambda b, kb: (b, 0, 0))],
        out_specs=pl.BlockSpec((1, 1, K), lambda b, kb: (b, 0, 0)),
        out_shape=jax.ShapeDtypeStruct((B, 1, K), jnp.float32),
        scratch_shapes=[pltpu.VMEM((1, K), jnp.float32)],
    )(icols, tblT)


_NC, _NS, _NW = 2, 16, 32
_SPAD = K + C
_B = 4
_ATR = 2 * NPAD

_sc_mesh = plsc.VectorSubcoreMesh(
    core_axis_name="c", subcore_axis_name="s", num_cores=_NC, num_subcores=_NS)

_SC_PARAMS = pltpu.CompilerParams(use_tc_tiling_on_sc=False,
                                  needs_layout_passes=False)


_CHA = NPAD // _NS


def _sc_compact_body(key_hbm, table_hbm, tk_hbm, ti_hbm, atbl_hbm, acnt_hbm,
                     keys_v, mask_v, rows_v, idx2d, tk_v, ti_v, acv_v,
                     ctr_smem, sem):
    cid = lax.axis_index("c")
    sid = lax.axis_index("s")

    @pl.when(sid == 0)
    def _():
        ctr_smem[0] = 0
        ctr_smem[1] = 0

    plsc.subcore_barrier()

    for b2 in range(2):
        b = cid * 2 + b2
        base = b * NPAD + sid * _CHA
        pltpu.sync_copy(key_hbm.at[pl.ds(base, _CHA)], keys_v)
        pltpu.sync_copy(tk_hbm.at[b], tk_v)
        pltpu.sync_copy(ti_hbm.at[b], ti_v)
        tk = jnp.max(tk_v[...])
        ti = jnp.max(ti_v[...])

        acc = jnp.zeros((16,), jnp.int32)
        for v in range(_CHA // 16):
            k16 = keys_v[pl.ds(v * 16, 16)]
            gi = sid * _CHA + v * 16 + lax.iota(jnp.int32, 16)
            m = (k16 > tk) | ((k16 == tk) & (gi <= ti))
            mask_v[pl.ds(v * 16, 16)] = jnp.where(m, 1, 0)
            acc = acc + jnp.where(m, 1, 0)
        cnt = jnp.sum(acc)

        base_slot = plsc.fetch_and_add(ctr_smem.at[b2], cnt, subcore_id=0)

        run = base_slot
        for v in range(_CHA // 16):
            mv = mask_v[pl.ds(v * 16, 16)]
            pref = plsc.cumsum(mv)
            slotg = sid * _CHA + v * 16 + lax.iota(jnp.int32, 16)
            dst = jnp.where(mv > 0, run + pref - 1, NPAD + slotg) + b * _ATR
            idx2d[v // 8, pl.ds((v % 8) * 16, 16)] = dst
            run = run + jnp.max(pref)

        pltpu.sync_copy(table_hbm.at[pl.ds(base, _CHA), :], rows_v)
        cps = [pltpu.async_copy(rows_v.at[pl.ds(j * 128, 128), :],
                                atbl_hbm.at[idx2d.at[j]], sem)
               for j in range(_CHA // 128)]
        for cp in cps:
            cp.wait()

    plsc.subcore_barrier()

    @pl.when(sid == 0)
    def _():
        for b2 in range(2):
            b = cid * 2 + b2
            acv_v[...] = jnp.zeros((16,), jnp.int32) + ctr_smem[b2]
            pltpu.sync_copy(acv_v, acnt_hbm.at[b])


def _stageA(key_flat, table_flat, tk16, ti16):
    return pl.kernel(
        _sc_compact_body,
        out_type=[jax.ShapeDtypeStruct((_B * _ATR, 16), jnp.float32),
                  jax.ShapeDtypeStruct((_B, 16), jnp.int32)],
        mesh=_sc_mesh,
        compiler_params=_SC_PARAMS,
        scratch_types=[pltpu.VMEM((_CHA,), jnp.int32),
                       pltpu.VMEM((_CHA,), jnp.int32),
                       pltpu.VMEM((_CHA, 16), jnp.float32),
                       pltpu.VMEM((_CHA // 128, 128), jnp.int32),
                       pltpu.VMEM((16,), jnp.int32),
                       pltpu.VMEM((16,), jnp.int32),
                       pltpu.VMEM((16,), jnp.int32),
                       pltpu.SMEM((8,), jnp.int32),
                       pltpu.SemaphoreType.DMA],
    )(key_flat, table_flat, tk16, ti16)


_CH3 = C // _NS


def _sc_scatter_body(rank_hbm, atbl_hbm, sorted_hbm, rank_v, rows_v, idx2d, sem):
    cid = lax.axis_index("c")
    sid = lax.axis_index("s")
    for b2 in range(2):
        b = cid * 2 + b2
        rbase = b * C + sid * _CH3
        tbase = b * _ATR + sid * _CH3
        pltpu.sync_copy(rank_hbm.at[pl.ds(rbase, _CH3)], rank_v)
        pltpu.sync_copy(atbl_hbm.at[pl.ds(tbase, _CH3), :], rows_v)
        for j in range(_CH3 // 128):
            for i in range(8):
                off = j * 128 + i * 16
                r = rank_v[pl.ds(off, 16)]
                trash = K + sid * _CH3 + off + lax.iota(jnp.int32, 16)
                idx2d[j, pl.ds(i * 16, 16)] = jnp.where(r < K, r, trash) + b * _SPAD
        cps = [pltpu.async_copy(rows_v.at[pl.ds(j * 128, 128), :],
                                sorted_hbm.at[idx2d.at[j]], sem)
               for j in range(_CH3 // 128)]
        for cp in cps:
            cp.wait()


def _stage3(rank_flat, atbl_flat):
    return pl.kernel(
        _sc_scatter_body,
        out_type=jax.ShapeDtypeStruct((_B * _SPAD, 16), jnp.float32),
        mesh=_sc_mesh,
        compiler_params=_SC_PARAMS,
        scratch_types=[pltpu.VMEM((_CH3,), jnp.int32),
                       pltpu.VMEM((_CH3, 16), jnp.float32),
                       pltpu.VMEM((_CH3 // 128, 128), jnp.int32),
                       pltpu.SemaphoreType.DMA],
    )(rank_flat, atbl_flat)


_OUTP = 512


def _sc_select_body(keep_hbm, sorted_hbm, fb_hbm, fs_hbm, fl_hbm,
                    keep_v, sel_v, idx2d, rows_v, sc_v, lb_v, sem):
    wid = lax.axis_index("s") * _NC + lax.axis_index("c")
    b = wid

    @pl.when(wid < _B)
    def _():
        pltpu.sync_copy(keep_hbm.at[pl.ds(b * K, K)], keep_v)

        def comp(i, cnt):
            kv = keep_v[pl.ds(i * 16, 16)]
            mask = kv > 0.5
            pref = plsc.cumsum(jnp.where(mask, 1, 0))
            posv = i * 16 + lax.iota(jnp.int32, 16)
            dst = jnp.where(mask, cnt + pref - 1, K + 8)
            plsc.store_scatter(sel_v, [dst], posv)
            return cnt + jnp.max(pref)

        count = lax.fori_loop(0, K // 16, comp, 0)

        for i in range(_OUTP // 16):
            sl = sel_v[pl.ds(i * 16, 16)]
            slot = i * 16 + lax.iota(jnp.int32, 16)
            idx = jnp.where(slot < count, sl, 0) + b * _SPAD
            idx2d[i // 8, pl.ds((i % 8) * 16, 16)] = idx

        cps = [pltpu.async_copy(sorted_hbm.at[idx2d.at[j]],
                                rows_v.at[pl.ds(j * 128, 128), :], sem)
               for j in range(_OUTP // 128)]
        for cp in cps:
            cp.wait()

        def zero(i, _):
            vf = jnp.where(i < count, 1.0, 0.0)
            rows_v[i] = rows_v[i] * vf
            return 0

        lax.fori_loop(0, _OUTP, zero, 0)

        def cols(i, _):
            slot = i * 16 + lax.iota(jnp.int32, 16)
            sc = plsc.load_gather(rows_v, [slot, jnp.full((16,), 7, jnp.int32)])
            lb = plsc.load_gather(rows_v, [slot, jnp.full((16,), 8, jnp.int32)])
            sc_v[pl.ds(i * 16, 16)] = sc
            lb_v[pl.ds(i * 16, 16)] = lb.astype(jnp.int32)
            return 0

        lax.fori_loop(0, _OUTP // 16, cols, 0)

        pltpu.sync_copy(rows_v, fb_hbm.at[pl.ds(b * _OUTP, _OUTP), :])
        pltpu.sync_copy(sc_v, fs_hbm.at[pl.ds(b * _OUTP, _OUTP)])
        pltpu.sync_copy(lb_v, fl_hbm.at[pl.ds(b * _OUTP, _OUTP)])


def _stage5(keep_flat, sorted_flat):
    return pl.kernel(
        _sc_select_body,
        out_type=[jax.ShapeDtypeStruct((_B * _OUTP, 16), jnp.float32),
                  jax.ShapeDtypeStruct((_B * _OUTP,), jnp.float32),
                  jax.ShapeDtypeStruct((_B * _OUTP,), jnp.int32)],
        mesh=_sc_mesh,
        compiler_params=pltpu.CompilerParams(use_tc_tiling_on_sc=False, needs_layout_passes=False),
        scratch_types=[pltpu.VMEM((K,), jnp.float32),
                       pltpu.VMEM((K + 16,), jnp.int32),
                       pltpu.VMEM((_OUTP // 128, 128), jnp.int32),
                       pltpu.VMEM((_OUTP, 16), jnp.float32),
                       pltpu.VMEM((_OUTP,), jnp.float32),
                       pltpu.VMEM((_OUTP,), jnp.int32),
                       pltpu.SemaphoreType.DMA],
    )(keep_flat, sorted_flat)


@jax.jit
def _run(batch_box_preds, batch_cls_preds):
    B, n, _ = batch_box_preds.shape
    cls3d = jnp.pad(batch_cls_preds, ((0, 0), (0, NPAD - n), (0, 0)),
                    constant_values=-30.0).transpose(0, 2, 1)
    boxes = jnp.pad(batch_box_preds, ((0, 0), (0, NPAD - n), (0, 0)))

    masked3, label3 = _stage1(cls3d)
    masked = masked3.reshape(B, NPAD)
    label = label3.reshape(B, NPAD)
    key = jax.lax.bitcast_convert_type(masked, jnp.int32)

    piv = key[:, ::PSTRIDE]
    rankp = _stage2a(piv)
    sel = rankp == S_SEL
    tkey = jnp.sum(jnp.where(sel, piv, 0), axis=1)
    tm = jnp.sum(jnp.where(sel, jnp.arange(NPIV, dtype=jnp.int32)[None, :], 0),
                 axis=1)
    tk16 = jnp.broadcast_to(tkey[:, None], (B, 16)).astype(jnp.int32)
    ti16 = jnp.broadcast_to((tm * PSTRIDE)[:, None], (B, 16)).astype(jnp.int32)

    oidx = jnp.broadcast_to(
        jnp.arange(NPAD, dtype=jnp.float32)[None, :, None], (B, NPAD, 1))
    table = jnp.concatenate(
        [boxes, masked[..., None], label.astype(jnp.float32)[..., None],
         oidx, jnp.zeros((B, NPAD, 6), jnp.float32)], axis=-1)

    atbl_flat, acnt = _stageA(key.reshape(B * NPAD),
                              table.reshape(B * NPAD, 16), tk16, ti16)
    atbl4 = atbl_flat.reshape(B, _ATR, 16)
    akey = jax.lax.bitcast_convert_type(atbl4[:, :C, 7], jnp.int32)
    aidx = atbl4[:, :C, 9].astype(jnp.int32)
    rankA = _stage2b(akey, aidx, acnt)

    sorted_flat = _stage3(rankA.reshape(B * C), atbl_flat)
    sorted4 = sorted_flat.reshape(B, _SPAD, 16)
    icols = sorted4[:, :K, :8]
    tblT = icols.transpose(0, 2, 1)

    keep3 = _stage4(icols, tblT)

    fbf, fsf, flf = _stage5(keep3.reshape(B * K), sorted_flat)
    fb = fbf.reshape(B, _OUTP, 16)[:, :OUT, :7]
    fs = fsf.reshape(B, _OUTP)[:, :OUT]
    fl = flf.reshape(B, _OUTP)[:, :OUT]
    return fb, fs, fl


def kernel(batch_box_preds, batch_cls_preds):
    return _run(batch_box_preds, batch_cls_preds)
